# Initial kernel scaffold; baseline (speedup 1.0000x reference)
import jax
import jax.numpy as jnp
from jax.experimental import pallas as pl


def kernel(x, edge_index_rel0, edge_index_rel1, edge_label_index, p0, Wih0, Whh0, bih0, bhh0, W0, p1, Wih1, Whh1, bih1, bhh1, W1, Wpost, bpost):
    raise NotImplementedError("write your pallas kernel here")


# speedup vs baseline: 8.5789x; 8.5789x over previous
import functools

import jax
import jax.numpy as jnp
from jax import lax
from jax.experimental import pallas as pl
from jax.experimental.pallas import tpu as pltpu
from jax.experimental.pallas import tpu_sc as plsc

N_NODES = 10000
FEAT = 256
N_EDGES = 160000
N_LABEL = 20000

NC = 2
NS = 16
NW = NC * NS

E_CHUNKS = 79
E_PER_TILE = E_CHUNKS * 128
E_PAD = NS * E_PER_TILE
ACC_ROWS = 10240
DEG_PER_TILE = ACC_ROWS // NS

L_CHUNKS = 5
L_PER_W = L_CHUNKS * 128
L_PAD = NW * L_PER_W

def _mesh():
    return plsc.VectorSubcoreMesh(core_axis_name="c", subcore_axis_name="s")


def _k1_body(x_ref, p_ref, s_ref):
    s_ref[...] = jnp.dot(x_ref[...], p_ref[...],
                         preferred_element_type=jnp.float32)


def _scores(x, p128):
    return pl.pallas_call(
        _k1_body,
        out_shape=jax.ShapeDtypeStruct((N_NODES, 128), jnp.float32),
    )(x, p128)


def _k1b_body(s0_ref, s1_ref, p2_ref, idx0_ref, idx1_ref, tv0_ref, tv1_ref):
    s0 = s0_ref[...]
    s1 = s1_ref[...]
    pp = p2_ref[...]
    nrm0 = jnp.sqrt(jnp.sum(pp[:, 0:1] * pp[:, 0:1]))
    nrm1 = jnp.sqrt(jnp.sum(pp[:, 1:2] * pp[:, 1:2]))
    inv0 = 1.0 / (nrm0 + 1e-16)
    inv1 = 1.0 / (nrm1 + 1e-16)

    fi = (lax.broadcasted_iota(jnp.int32, (80, 128), 0) * 128
          + lax.broadcasted_iota(jnp.int32, (80, 128), 1))
    i256 = lax.iota(jnp.int32, 256)
    neg = jnp.float32(-jnp.inf)
    big = jnp.int32(1 << 30)

    def step(r, carry):
        c0, c1, va0, va1, ia0, ia1 = carry

        m0 = jnp.max(c0)
        ix0 = jnp.min(jnp.where(c0 == m0, fi, big))
        c0 = jnp.where(fi == ix0, neg, c0)
        va0 = jnp.where(i256 == r, m0, va0)
        ia0 = jnp.where(i256 == r, ix0, ia0)

        m1 = jnp.max(c1)
        ix1 = jnp.min(jnp.where(c1 == m1, fi, big))
        c1 = jnp.where(fi == ix1, neg, c1)
        va1 = jnp.where(i256 == r, m1, va1)
        ia1 = jnp.where(i256 == r, ix1, ia1)

        return c0, c1, va0, va1, ia0, ia1

    init = (s0, s1,
            jnp.zeros((256,), jnp.float32), jnp.zeros((256,), jnp.float32),
            jnp.zeros((256,), jnp.int32), jnp.zeros((256,), jnp.int32))
    _, _, va0, va1, ia0, ia1 = lax.fori_loop(0, 256, step, init)

    idx0_ref[...] = ia0
    idx1_ref[...] = ia1
    tv0_ref[...] = jnp.tanh(va0 * inv0)
    tv1_ref[...] = jnp.tanh(va1 * inv1)


def _topk(s0, s1, p2):
    return pl.pallas_call(
        _k1b_body,
        out_shape=(
            jax.ShapeDtypeStruct((256,), jnp.int32),
            jax.ShapeDtypeStruct((256,), jnp.int32),
            jax.ShapeDtypeStruct((256,), jnp.float32),
            jax.ShapeDtypeStruct((256,), jnp.float32),
        ),
    )(s0, s1, p2)


def _k2_body(dst0_ref, dst1_ref, idx0_ref, idx1_ref, x_ref,
             deg0_ref, deg1_ref, xt0_ref, xt1_ref,
             idx_v, ones_v, gidx_v, rows_v, zrow_v, deg_sh, sem):
    cid = lax.axis_index("c")
    sid = lax.axis_index("s")

    z16 = jnp.zeros((16,), jnp.float32)
    o16 = jnp.ones((16,), jnp.float32)

    def zb(k, _):
        zrow_v[pl.ds(k * 16, 16)] = z16
        return 0
    lax.fori_loop(0, DEG_PER_TILE // 16, zb, 0)
    for k in range(8):
        ones_v[pl.ds(k * 16, 16)] = o16

    pltpu.sync_copy(zrow_v, deg_sh.at[pl.ds(sid * DEG_PER_TILE, DEG_PER_TILE)])
    plsc.subcore_barrier()

    @pl.when(cid == 0)
    def _():
        pltpu.sync_copy(dst0_ref.at[sid], idx_v)

    @pl.when(cid == 1)
    def _():
        pltpu.sync_copy(dst1_ref.at[sid], idx_v)

    def add1(j, _):
        pltpu.sync_copy(ones_v, deg_sh.at[idx_v.at[j]], add=True)
        return 0
    lax.fori_loop(0, E_CHUNKS, add1, 0)
    plsc.subcore_barrier()

    @pl.when(cid == 0)
    def _():
        pltpu.sync_copy(deg_sh.at[pl.ds(sid * DEG_PER_TILE, DEG_PER_TILE)],
                        deg0_ref.at[pl.ds(sid * DEG_PER_TILE, DEG_PER_TILE)])

    @pl.when(cid == 1)
    def _():
        pltpu.sync_copy(deg_sh.at[pl.ds(sid * DEG_PER_TILE, DEG_PER_TILE)],
                        deg1_ref.at[pl.ds(sid * DEG_PER_TILE, DEG_PER_TILE)])

    @pl.when(cid == 0)
    def _():
        pltpu.sync_copy(idx0_ref.at[pl.ds(sid * 16, 16)], gidx_v)
        pltpu.async_copy(x_ref.at[gidx_v], rows_v, sem).wait()
        pltpu.sync_copy(rows_v, xt0_ref.at[pl.ds(sid * 16, 16)])

    @pl.when(cid == 1)
    def _():
        pltpu.sync_copy(idx1_ref.at[pl.ds(sid * 16, 16)], gidx_v)
        pltpu.async_copy(x_ref.at[gidx_v], rows_v, sem).wait()
        pltpu.sync_copy(rows_v, xt1_ref.at[pl.ds(sid * 16, 16)])


def _deg_and_gather(dst0p, dst1p, idx0, idx1, x):
    f = functools.partial(
        pl.kernel,
        out_type=(
            jax.ShapeDtypeStruct((ACC_ROWS,), jnp.float32),
            jax.ShapeDtypeStruct((ACC_ROWS,), jnp.float32),
            jax.ShapeDtypeStruct((256, FEAT), jnp.float32),
            jax.ShapeDtypeStruct((256, FEAT), jnp.float32),
        ),
        mesh=_mesh(),
        scratch_types=[
            pltpu.VMEM((E_CHUNKS, 128), jnp.int32),
            pltpu.VMEM((128,), jnp.float32),
            pltpu.VMEM((16,), jnp.int32),
            pltpu.VMEM((16, FEAT), jnp.float32),
            pltpu.VMEM((DEG_PER_TILE,), jnp.float32),
            pltpu.VMEM_SHARED((ACC_ROWS,), jnp.float32),
            pltpu.SemaphoreType.DMA,
        ],
    )
    return f(_k2_body)(dst0p, dst1p, idx0, idx1, x)


def _gru_half(xt, tv, wiht, whht, bih, bhh, w0):
    xts = xt * tv
    gi = jnp.dot(xts, wiht, preferred_element_type=jnp.float32) + bih[None, :]
    gh = jnp.dot(w0, whht, preferred_element_type=jnp.float32) + bhh[None, :]
    i_r, i_z, i_n = gi[:, :256], gi[:, 256:512], gi[:, 512:]
    h_r, h_z, h_n = gh[:, :256], gh[:, 256:512], gh[:, 512:]
    r = 1.0 / (1.0 + jnp.exp(-(i_r + h_r)))
    z = 1.0 / (1.0 + jnp.exp(-(i_z + h_z)))
    n = jnp.tanh(i_n + r * h_n)
    return (1.0 - z) * n + z * w0


def _k3a_body(xt0_ref, tv0_ref, wiht0_ref, whht0_ref, bih0_ref, bhh0_ref,
              w00_ref, xt1_ref, tv1_ref, wiht1_ref, whht1_ref, bih1_ref,
              bhh1_ref, w01_ref, wm0_ref, wm1_ref):
    wm0_ref[...] = _gru_half(xt0_ref[...], tv0_ref[...], wiht0_ref[...],
                             whht0_ref[...], bih0_ref[...], bhh0_ref[...],
                             w00_ref[...])
    wm1_ref[...] = _gru_half(xt1_ref[...], tv1_ref[...], wiht1_ref[...],
                             whht1_ref[...], bih1_ref[...], bhh1_ref[...],
                             w01_ref[...])


def _gru(xt0, tv0, wiht0, whht0, bih0, bhh0, w00,
         xt1, tv1, wiht1, whht1, bih1, bhh1, w01):
    return pl.pallas_call(
        _k3a_body,
        out_shape=(
            jax.ShapeDtypeStruct((FEAT, FEAT), jnp.float32),
            jax.ShapeDtypeStruct((FEAT, FEAT), jnp.float32),
        ),
    )(xt0, tv0, wiht0, whht0, bih0, bhh0, w00,
      xt1, tv1, wiht1, whht1, bih1, bhh1, w01)


_RB = 2000
NQ = 4
QW = FEAT // NQ


def _k3b_body(x_ref, wm0_ref, wm1_ref, deg0_ref, deg1_ref, *a_refs):
    xb = x_ref[...]
    d0 = lax.rsqrt(deg0_ref[...] + 1.0)
    d1 = lax.rsqrt(deg1_ref[...] + 1.0)
    xs0 = jnp.dot(xb, wm0_ref[...], preferred_element_type=jnp.float32) * d0
    xs1 = jnp.dot(xb, wm1_ref[...], preferred_element_type=jnp.float32) * d1
    for q in range(NQ):
        a_refs[q][...] = xs0[:, q * QW:(q + 1) * QW]
        a_refs[NQ + q][...] = xs1[:, q * QW:(q + 1) * QW]


def _xws(x, wm0, wm1, deg0c, deg1c):
    nblk = N_NODES // _RB
    quarter = jax.ShapeDtypeStruct((N_NODES, QW), jnp.float32)
    return pl.pallas_call(
        _k3b_body,
        grid=(nblk,),
        in_specs=[
            pl.BlockSpec((_RB, FEAT), lambda i: (i, 0)),
            pl.BlockSpec((FEAT, FEAT), lambda i: (0, 0)),
            pl.BlockSpec((FEAT, FEAT), lambda i: (0, 0)),
            pl.BlockSpec((_RB, 1), lambda i: (i, 0)),
            pl.BlockSpec((_RB, 1), lambda i: (i, 0)),
        ],
        out_specs=(pl.BlockSpec((_RB, QW), lambda i: (i, 0)),) * (2 * NQ),
        out_shape=(quarter,) * (2 * NQ),
    )(x, wm0, wm1, deg0c, deg1c)


def _k4_body(src0_ref, dst0_ref, src1_ref, dst1_ref, *refs):
    a_refs = refs[:2 * NQ]
    o_refs = refs[2 * NQ:4 * NQ]
    sidx_v, didx_v, rows_v, zbuf_v, acc_sh, sem = refs[4 * NQ:]
    cid = lax.axis_index("c")
    sid = lax.axis_index("s")

    z16 = jnp.zeros((16,), jnp.float32)

    def zb(i, _):
        for k in range(QW // 16):
            zbuf_v[i, pl.ds(k * 16, 16)] = z16
        return 0
    lax.fori_loop(0, 128, zb, 0)

    @pl.when(cid == 0)
    def _():
        pltpu.sync_copy(src0_ref.at[sid], sidx_v)
        pltpu.sync_copy(dst0_ref.at[sid], didx_v)

    @pl.when(cid == 1)
    def _():
        pltpu.sync_copy(src1_ref.at[sid], sidx_v)
        pltpu.sync_copy(dst1_ref.at[sid], didx_v)

    def do_quarter(a_ref, o_ref):
        for j in range(DEG_PER_TILE // 128):
            pltpu.sync_copy(
                zbuf_v,
                acc_sh.at[pl.ds(sid * DEG_PER_TILE + j * 128, 128)])
        plsc.subcore_barrier()

        def step(j, _):
            pltpu.async_copy(a_ref.at[sidx_v.at[j]], rows_v, sem).wait()
            pltpu.sync_copy(rows_v, acc_sh.at[didx_v.at[j]], add=True)
            return 0
        lax.fori_loop(0, E_CHUNKS, step, 0)
        plsc.subcore_barrier()

        for j in range(DEG_PER_TILE // 128):
            s = pl.ds(sid * DEG_PER_TILE + j * 128, 128)
            pltpu.sync_copy(acc_sh.at[s], o_ref.at[s])
        plsc.subcore_barrier()

    @pl.when(cid == 0)
    def _():
        for q in range(NQ):
            do_quarter(a_refs[q], o_refs[q])

    @pl.when(cid == 1)
    def _():
        for q in range(NQ):
            do_quarter(a_refs[NQ + q], o_refs[NQ + q])


def _message_passing(src0p, dst0p, src1p, dst1p, aq):
    acc = jax.ShapeDtypeStruct((ACC_ROWS, QW), jnp.float32)
    f = functools.partial(
        pl.kernel,
        out_type=(acc,) * (2 * NQ),
        mesh=_mesh(),
        scratch_types=[
            pltpu.VMEM((E_CHUNKS, 128), jnp.int32),
            pltpu.VMEM((E_CHUNKS, 128), jnp.int32),
            pltpu.VMEM((128, QW), jnp.float32),
            pltpu.VMEM((128, QW), jnp.float32),
            pltpu.VMEM_SHARED((ACC_ROWS, QW), jnp.float32),
            pltpu.SemaphoreType.DMA,
        ],
        compiler_params=pltpu.CompilerParams(use_tc_tiling_on_sc=False, needs_layout_passes=False),
    )
    return f(_k4_body)(src0p, dst0p, src1p, dst1p, *aq)


def _k5_body(*refs):
    o_refs = refs[:2 * NQ]
    a_refs = refs[2 * NQ:4 * NQ]
    deg0_ref, deg1_ref, wpt_ref, h_ref, hw_ref = refs[4 * NQ:]
    d0 = lax.rsqrt(deg0_ref[...] + 1.0)
    d1 = lax.rsqrt(deg1_ref[...] + 1.0)
    parts = []
    for q in range(NQ):
        parts.append(d0 * (o_refs[q][...] + a_refs[q][...])
                     + d1 * (o_refs[NQ + q][...] + a_refs[NQ + q][...]))
    h = jnp.maximum(jnp.concatenate(parts, axis=1), 0.0)
    wsum = wpt_ref[0:1, :] + wpt_ref[1:2, :]
    h_ref[...] = h
    hw_ref[...] = h * wsum


def _combine(oq, aq, deg0c, deg1c, wpostt):
    nblk = N_NODES // _RB
    ob = pl.BlockSpec((_RB, QW), lambda i: (i, 0))
    return pl.pallas_call(
        _k5_body,
        grid=(nblk,),
        in_specs=[ob] * (4 * NQ)
        + [pl.BlockSpec((_RB, 1), lambda i: (i, 0)),
           pl.BlockSpec((_RB, 1), lambda i: (i, 0)),
           pl.BlockSpec((2, FEAT), lambda i: (0, 0))],
        out_specs=(pl.BlockSpec((_RB, FEAT), lambda i: (i, 0)),
                   pl.BlockSpec((_RB, FEAT), lambda i: (i, 0))),
        out_shape=(jax.ShapeDtypeStruct((N_NODES, FEAT), jnp.float32),
                   jax.ShapeDtypeStruct((N_NODES, FEAT), jnp.float32)),
    )(*oq, *aq, deg0c, deg1c, wpostt)


def _k6_body(h_ref, hw_ref, la_ref, lb_ref, bp_ref, res_ref,
             ia_v, ib_v, ga_v, gb_v, res_v, bp_v, sem):
    cid = lax.axis_index("c")
    sid = lax.axis_index("s")
    w = sid * NC + cid

    pltpu.sync_copy(la_ref.at[w], ia_v)
    pltpu.sync_copy(lb_ref.at[w], ib_v)
    pltpu.sync_copy(bp_ref, bp_v)
    c = jnp.sum(bp_v[...])
    i16 = lax.iota(jnp.int32, 16)

    for j in range(L_CHUNKS):
        pltpu.async_copy(h_ref.at[ia_v.at[j]], ga_v, sem).wait()
        pltpu.async_copy(hw_ref.at[ib_v.at[j]], gb_v, sem).wait()

        def grp(g, _):
            vec = jnp.zeros((16,), jnp.float32)
            for i in range(16):
                p = g * 16 + i
                acc = jnp.zeros((16,), jnp.float32)
                for k in range(16):
                    acc = acc + (ga_v[p, pl.ds(k * 16, 16)]
                                 * gb_v[p, pl.ds(k * 16, 16)])
                s = jnp.sum(acc)
                vec = jnp.where(i16 == i, s, vec)
            res_v[pl.ds(j * 128 + g * 16, 16)] = vec + c
            return 0
        lax.fori_loop(0, 8, grp, 0)

    pltpu.sync_copy(res_v, res_ref.at[w])


def _label_score(h, hw, la, lb, bp16):
    f = functools.partial(
        pl.kernel,
        out_type=jax.ShapeDtypeStruct((NW, L_PER_W), jnp.float32),
        mesh=_mesh(),
        scratch_types=[
            pltpu.VMEM((L_CHUNKS, 128), jnp.int32),
            pltpu.VMEM((L_CHUNKS, 128), jnp.int32),
            pltpu.VMEM((128, FEAT), jnp.float32),
            pltpu.VMEM((128, FEAT), jnp.float32),
            pltpu.VMEM((L_PER_W,), jnp.float32),
            pltpu.VMEM((16,), jnp.float32),
            pltpu.SemaphoreType.DMA,
        ],
        compiler_params=pltpu.CompilerParams(needs_layout_passes=False),
    )
    return f(_k6_body)(h, hw, la, lb, bp16)


def kernel(x, edge_index_rel0, edge_index_rel1, edge_label_index,
           p0, Wih0, Whh0, bih0, bhh0, W0,
           p1, Wih1, Whh1, bih1, bhh1, W1,
           Wpost, bpost):
    p128 = jnp.pad(jnp.stack([p0, p1], axis=1), ((0, 0), (0, 126)))
    p2 = jnp.stack([p0, p1], axis=1)

    npad = E_PAD - N_EDGES
    ar = lax.iota(jnp.int32, npad)
    sent_src = ar % 64
    sent_dst = N_NODES + (ar % 128)

    def prep_edges(ei):
        s = jnp.concatenate([ei[0], sent_src]).reshape(NS, E_CHUNKS, 128)
        d = jnp.concatenate([ei[1], sent_dst]).reshape(NS, E_CHUNKS, 128)
        return s, d

    src0p, dst0p = prep_edges(edge_index_rel0)
    src1p, dst1p = prep_edges(edge_index_rel1)

    lpad = L_PAD - N_LABEL
    lar = lax.iota(jnp.int32, lpad) % 64
    la = jnp.concatenate([edge_label_index[0], lar]).reshape(NW, L_CHUNKS, 128)
    lb = jnp.concatenate([edge_label_index[1], lar]).reshape(NW, L_CHUNKS, 128)

    wiht0, whht0 = Wih0.T, Whh0.T
    wiht1, whht1 = Wih1.T, Whh1.T
    wpostt = Wpost.T
    bp16 = jnp.pad(bpost, (0, 14))

    S = _scores(x, p128)
    spad = jnp.pad(S[:, :2], ((0, 240), (0, 0)),
                   constant_values=-jnp.inf)
    s0 = spad[:, 0].reshape(80, 128)
    s1 = spad[:, 1].reshape(80, 128)
    idx0, idx1, tv0, tv1 = _topk(s0, s1, p2)

    deg0, deg1, xt0, xt1 = _deg_and_gather(dst0p, dst1p, idx0, idx1, x)
    deg0c = deg0[:N_NODES].reshape(N_NODES, 1)
    deg1c = deg1[:N_NODES].reshape(N_NODES, 1)

    wm0, wm1 = _gru(xt0, tv0.reshape(256, 1), wiht0, whht0, bih0, bhh0, W0,
                    xt1, tv1.reshape(256, 1), wiht1, whht1, bih1, bhh1, W1)
    aq = _xws(x, wm0, wm1, deg0c, deg1c)

    oq = _message_passing(src0p, dst0p, src1p, dst1p, aq)

    h, hw = _combine([o[:N_NODES] for o in oq], aq, deg0c, deg1c, wpostt)

    res = _label_score(h, hw, la, lb, bp16)
    return res.reshape(L_PAD)[:N_LABEL]


# speedup vs baseline: 9.8148x; 1.1441x over previous
import functools

import jax
import jax.numpy as jnp
from jax import lax
from jax.experimental import pallas as pl
from jax.experimental.pallas import tpu as pltpu
from jax.experimental.pallas import tpu_sc as plsc

N_NODES = 10000
FEAT = 256
N_EDGES = 160000
N_LABEL = 20000

NC = 2
NS = 16
NW = NC * NS

E_CHUNKS = 79
E_PER_TILE = E_CHUNKS * 128
E_PAD = NS * E_PER_TILE
ACC_ROWS = 10240
DEG_PER_TILE = ACC_ROWS // NS

L_CHUNKS = 5
L_PER_W = L_CHUNKS * 128
L_PAD = NW * L_PER_W

def _mesh():
    return plsc.VectorSubcoreMesh(core_axis_name="c", subcore_axis_name="s")


def _k1_body(x_ref, p_ref, s_ref):
    s_ref[...] = jnp.dot(x_ref[...], p_ref[...],
                         preferred_element_type=jnp.float32)


def _scores(x, p128):
    return pl.pallas_call(
        _k1_body,
        out_shape=jax.ShapeDtypeStruct((N_NODES, 128), jnp.float32),
    )(x, p128)


def _k1b_body(s0_ref, s1_ref, p2_ref, idx0_ref, idx1_ref, tv0_ref, tv1_ref):
    s0 = s0_ref[...]
    s1 = s1_ref[...]
    pp = p2_ref[...]
    nrm0 = jnp.sqrt(jnp.sum(pp[:, 0:1] * pp[:, 0:1]))
    nrm1 = jnp.sqrt(jnp.sum(pp[:, 1:2] * pp[:, 1:2]))
    inv0 = 1.0 / (nrm0 + 1e-16)
    inv1 = 1.0 / (nrm1 + 1e-16)

    fi = (lax.broadcasted_iota(jnp.int32, (80, 128), 0) * 128
          + lax.broadcasted_iota(jnp.int32, (80, 128), 1))
    i256 = lax.iota(jnp.int32, 256)
    neg = jnp.float32(-jnp.inf)
    big = jnp.int32(1 << 30)

    def step(r, carry):
        c0, c1, va0, va1, ia0, ia1 = carry

        m0 = jnp.max(c0)
        ix0 = jnp.min(jnp.where(c0 == m0, fi, big))
        c0 = jnp.where(fi == ix0, neg, c0)
        va0 = jnp.where(i256 == r, m0, va0)
        ia0 = jnp.where(i256 == r, ix0, ia0)

        m1 = jnp.max(c1)
        ix1 = jnp.min(jnp.where(c1 == m1, fi, big))
        c1 = jnp.where(fi == ix1, neg, c1)
        va1 = jnp.where(i256 == r, m1, va1)
        ia1 = jnp.where(i256 == r, ix1, ia1)

        return c0, c1, va0, va1, ia0, ia1

    init = (s0, s1,
            jnp.zeros((256,), jnp.float32), jnp.zeros((256,), jnp.float32),
            jnp.zeros((256,), jnp.int32), jnp.zeros((256,), jnp.int32))
    _, _, va0, va1, ia0, ia1 = lax.fori_loop(0, 256, step, init)

    idx0_ref[...] = ia0
    idx1_ref[...] = ia1
    tv0_ref[...] = jnp.tanh(va0 * inv0)
    tv1_ref[...] = jnp.tanh(va1 * inv1)


def _topk(s0, s1, p2):
    return pl.pallas_call(
        _k1b_body,
        out_shape=(
            jax.ShapeDtypeStruct((256,), jnp.int32),
            jax.ShapeDtypeStruct((256,), jnp.int32),
            jax.ShapeDtypeStruct((256,), jnp.float32),
            jax.ShapeDtypeStruct((256,), jnp.float32),
        ),
    )(s0, s1, p2)


def _k2_body(dst0_ref, dst1_ref, idx0_ref, idx1_ref, x_ref,
             deg0_ref, deg1_ref, xt0_ref, xt1_ref,
             idx_v, ones_v, gidx_v, rows_v, zrow_v, deg_sh, sem):
    cid = lax.axis_index("c")
    sid = lax.axis_index("s")

    z16 = jnp.zeros((16,), jnp.float32)
    o16 = jnp.ones((16,), jnp.float32)

    def zb(k, _):
        zrow_v[pl.ds(k * 16, 16)] = z16
        return 0
    lax.fori_loop(0, DEG_PER_TILE // 16, zb, 0)
    for k in range(8):
        ones_v[pl.ds(k * 16, 16)] = o16

    pltpu.sync_copy(zrow_v, deg_sh.at[pl.ds(sid * DEG_PER_TILE, DEG_PER_TILE)])
    plsc.subcore_barrier()

    @pl.when(cid == 0)
    def _():
        pltpu.sync_copy(dst0_ref.at[sid], idx_v)

    @pl.when(cid == 1)
    def _():
        pltpu.sync_copy(dst1_ref.at[sid], idx_v)

    def add1(j, _):
        pltpu.sync_copy(ones_v, deg_sh.at[idx_v.at[j]], add=True)
        return 0
    lax.fori_loop(0, E_CHUNKS, add1, 0)
    plsc.subcore_barrier()

    @pl.when(cid == 0)
    def _():
        pltpu.sync_copy(deg_sh.at[pl.ds(sid * DEG_PER_TILE, DEG_PER_TILE)],
                        deg0_ref.at[pl.ds(sid * DEG_PER_TILE, DEG_PER_TILE)])

    @pl.when(cid == 1)
    def _():
        pltpu.sync_copy(deg_sh.at[pl.ds(sid * DEG_PER_TILE, DEG_PER_TILE)],
                        deg1_ref.at[pl.ds(sid * DEG_PER_TILE, DEG_PER_TILE)])

    @pl.when(cid == 0)
    def _():
        pltpu.sync_copy(idx0_ref.at[pl.ds(sid * 16, 16)], gidx_v)
        pltpu.async_copy(x_ref.at[gidx_v], rows_v, sem).wait()
        pltpu.sync_copy(rows_v, xt0_ref.at[pl.ds(sid * 16, 16)])

    @pl.when(cid == 1)
    def _():
        pltpu.sync_copy(idx1_ref.at[pl.ds(sid * 16, 16)], gidx_v)
        pltpu.async_copy(x_ref.at[gidx_v], rows_v, sem).wait()
        pltpu.sync_copy(rows_v, xt1_ref.at[pl.ds(sid * 16, 16)])


def _deg_and_gather(dst0p, dst1p, idx0, idx1, x):
    f = functools.partial(
        pl.kernel,
        out_type=(
            jax.ShapeDtypeStruct((ACC_ROWS,), jnp.float32),
            jax.ShapeDtypeStruct((ACC_ROWS,), jnp.float32),
            jax.ShapeDtypeStruct((256, FEAT), jnp.float32),
            jax.ShapeDtypeStruct((256, FEAT), jnp.float32),
        ),
        mesh=_mesh(),
        scratch_types=[
            pltpu.VMEM((E_CHUNKS, 128), jnp.int32),
            pltpu.VMEM((128,), jnp.float32),
            pltpu.VMEM((16,), jnp.int32),
            pltpu.VMEM((16, FEAT), jnp.float32),
            pltpu.VMEM((DEG_PER_TILE,), jnp.float32),
            pltpu.VMEM_SHARED((ACC_ROWS,), jnp.float32),
            pltpu.SemaphoreType.DMA,
        ],
    )
    return f(_k2_body)(dst0p, dst1p, idx0, idx1, x)


def _gru_half(xt, tv, wiht, whht, bih, bhh, w0):
    xts = xt * tv
    gi = jnp.dot(xts, wiht, preferred_element_type=jnp.float32) + bih[None, :]
    gh = jnp.dot(w0, whht, preferred_element_type=jnp.float32) + bhh[None, :]
    i_r, i_z, i_n = gi[:, :256], gi[:, 256:512], gi[:, 512:]
    h_r, h_z, h_n = gh[:, :256], gh[:, 256:512], gh[:, 512:]
    r = 1.0 / (1.0 + jnp.exp(-(i_r + h_r)))
    z = 1.0 / (1.0 + jnp.exp(-(i_z + h_z)))
    n = jnp.tanh(i_n + r * h_n)
    return (1.0 - z) * n + z * w0


def _k3a_body(xt0_ref, tv0_ref, wiht0_ref, whht0_ref, bih0_ref, bhh0_ref,
              w00_ref, xt1_ref, tv1_ref, wiht1_ref, whht1_ref, bih1_ref,
              bhh1_ref, w01_ref, wm0_ref, wm1_ref):
    wm0_ref[...] = _gru_half(xt0_ref[...], tv0_ref[...], wiht0_ref[...],
                             whht0_ref[...], bih0_ref[...], bhh0_ref[...],
                             w00_ref[...])
    wm1_ref[...] = _gru_half(xt1_ref[...], tv1_ref[...], wiht1_ref[...],
                             whht1_ref[...], bih1_ref[...], bhh1_ref[...],
                             w01_ref[...])


def _gru(xt0, tv0, wiht0, whht0, bih0, bhh0, w00,
         xt1, tv1, wiht1, whht1, bih1, bhh1, w01):
    return pl.pallas_call(
        _k3a_body,
        out_shape=(
            jax.ShapeDtypeStruct((FEAT, FEAT), jnp.float32),
            jax.ShapeDtypeStruct((FEAT, FEAT), jnp.float32),
        ),
    )(xt0, tv0, wiht0, whht0, bih0, bhh0, w00,
      xt1, tv1, wiht1, whht1, bih1, bhh1, w01)


_RB = 2000
NQ = 4
QW = FEAT // NQ


def _k3b_body(x_ref, wm0_ref, wm1_ref, deg0_ref, deg1_ref, *a_refs):
    xb = x_ref[...]
    d0 = lax.rsqrt(deg0_ref[...] + 1.0)
    d1 = lax.rsqrt(deg1_ref[...] + 1.0)
    xs0 = jnp.dot(xb, wm0_ref[...], preferred_element_type=jnp.float32) * d0
    xs1 = jnp.dot(xb, wm1_ref[...], preferred_element_type=jnp.float32) * d1
    for q in range(NQ):
        a_refs[q][...] = xs0[:, q * QW:(q + 1) * QW]
        a_refs[NQ + q][...] = xs1[:, q * QW:(q + 1) * QW]


def _xws(x, wm0, wm1, deg0c, deg1c):
    nblk = N_NODES // _RB
    quarter = jax.ShapeDtypeStruct((N_NODES, QW), jnp.float32)
    return pl.pallas_call(
        _k3b_body,
        grid=(nblk,),
        in_specs=[
            pl.BlockSpec((_RB, FEAT), lambda i: (i, 0)),
            pl.BlockSpec((FEAT, FEAT), lambda i: (0, 0)),
            pl.BlockSpec((FEAT, FEAT), lambda i: (0, 0)),
            pl.BlockSpec((_RB, 1), lambda i: (i, 0)),
            pl.BlockSpec((_RB, 1), lambda i: (i, 0)),
        ],
        out_specs=(pl.BlockSpec((_RB, QW), lambda i: (i, 0)),) * (2 * NQ),
        out_shape=(quarter,) * (2 * NQ),
    )(x, wm0, wm1, deg0c, deg1c)


def _k4_body(src0_ref, dst0_ref, src1_ref, dst1_ref, *refs):
    a_refs = refs[:2 * NQ]
    o_refs = refs[2 * NQ:4 * NQ]
    sidx_v, didx_v, rows0_v, rows1_v, zbuf_v, acc_sh, sem0, sem1 = refs[4 * NQ:]
    cid = lax.axis_index("c")
    sid = lax.axis_index("s")

    z16 = jnp.zeros((16,), jnp.float32)

    def zb(i, _):
        for k in range(QW // 16):
            zbuf_v[i, pl.ds(k * 16, 16)] = z16
        return 0
    lax.fori_loop(0, 128, zb, 0)

    @pl.when(cid == 0)
    def _():
        pltpu.sync_copy(src0_ref.at[sid], sidx_v)
        pltpu.sync_copy(dst0_ref.at[sid], didx_v)

    @pl.when(cid == 1)
    def _():
        pltpu.sync_copy(src1_ref.at[sid], sidx_v)
        pltpu.sync_copy(dst1_ref.at[sid], didx_v)

    def do_quarter(a_ref, o_ref):
        for j in range(DEG_PER_TILE // 128):
            pltpu.sync_copy(
                zbuf_v,
                acc_sh.at[pl.ds(sid * DEG_PER_TILE + j * 128, 128)])
        plsc.subcore_barrier()

        pltpu.async_copy(a_ref.at[sidx_v.at[0]], rows0_v, sem0)

        def step(j, _):
            @pl.when(j % 2 == 0)
            def _():
                pltpu.make_async_copy(
                    a_ref.at[sidx_v.at[j]], rows0_v, sem0).wait()

                @pl.when(j < E_CHUNKS - 1)
                def _():
                    pltpu.async_copy(
                        a_ref.at[sidx_v.at[j + 1]], rows1_v, sem1)
                pltpu.sync_copy(rows0_v, acc_sh.at[didx_v.at[j]], add=True)

            @pl.when(j % 2 == 1)
            def _():
                pltpu.make_async_copy(
                    a_ref.at[sidx_v.at[j]], rows1_v, sem1).wait()

                @pl.when(j < E_CHUNKS - 1)
                def _():
                    pltpu.async_copy(
                        a_ref.at[sidx_v.at[j + 1]], rows0_v, sem0)
                pltpu.sync_copy(rows1_v, acc_sh.at[didx_v.at[j]], add=True)
            return 0
        lax.fori_loop(0, E_CHUNKS, step, 0)
        plsc.subcore_barrier()

        for j in range(DEG_PER_TILE // 128):
            s = pl.ds(sid * DEG_PER_TILE + j * 128, 128)
            pltpu.sync_copy(acc_sh.at[s], o_ref.at[s])
        plsc.subcore_barrier()

    @pl.when(cid == 0)
    def _():
        for q in range(NQ):
            do_quarter(a_refs[q], o_refs[q])

    @pl.when(cid == 1)
    def _():
        for q in range(NQ):
            do_quarter(a_refs[NQ + q], o_refs[NQ + q])


def _message_passing(src0p, dst0p, src1p, dst1p, aq):
    acc = jax.ShapeDtypeStruct((ACC_ROWS, QW), jnp.float32)
    f = functools.partial(
        pl.kernel,
        out_type=(acc,) * (2 * NQ),
        mesh=_mesh(),
        scratch_types=[
            pltpu.VMEM((E_CHUNKS, 128), jnp.int32),
            pltpu.VMEM((E_CHUNKS, 128), jnp.int32),
            pltpu.VMEM((128, QW), jnp.float32),
            pltpu.VMEM((128, QW), jnp.float32),
            pltpu.VMEM((128, QW), jnp.float32),
            pltpu.VMEM_SHARED((ACC_ROWS, QW), jnp.float32),
            pltpu.SemaphoreType.DMA,
            pltpu.SemaphoreType.DMA,
        ],
        compiler_params=pltpu.CompilerParams(use_tc_tiling_on_sc=False, needs_layout_passes=False),
    )
    return f(_k4_body)(src0p, dst0p, src1p, dst1p, *aq)


def _k5_body(*refs):
    o_refs = refs[:2 * NQ]
    a_refs = refs[2 * NQ:4 * NQ]
    deg0_ref, deg1_ref, wpt_ref, h_ref, hw_ref = refs[4 * NQ:]
    d0 = lax.rsqrt(deg0_ref[...] + 1.0)
    d1 = lax.rsqrt(deg1_ref[...] + 1.0)
    parts = []
    for q in range(NQ):
        parts.append(d0 * (o_refs[q][...] + a_refs[q][...])
                     + d1 * (o_refs[NQ + q][...] + a_refs[NQ + q][...]))
    h = jnp.maximum(jnp.concatenate(parts, axis=1), 0.0)
    wsum = wpt_ref[0:1, :] + wpt_ref[1:2, :]
    h_ref[...] = h
    hw_ref[...] = h * wsum


def _combine(oq, aq, deg0c, deg1c, wpostt):
    nblk = N_NODES // _RB
    ob = pl.BlockSpec((_RB, QW), lambda i: (i, 0))
    return pl.pallas_call(
        _k5_body,
        grid=(nblk,),
        in_specs=[ob] * (4 * NQ)
        + [pl.BlockSpec((_RB, 1), lambda i: (i, 0)),
           pl.BlockSpec((_RB, 1), lambda i: (i, 0)),
           pl.BlockSpec((2, FEAT), lambda i: (0, 0))],
        out_specs=(pl.BlockSpec((_RB, FEAT), lambda i: (i, 0)),
                   pl.BlockSpec((_RB, FEAT), lambda i: (i, 0))),
        out_shape=(jax.ShapeDtypeStruct((N_NODES, FEAT), jnp.float32),
                   jax.ShapeDtypeStruct((N_NODES, FEAT), jnp.float32)),
    )(*oq, *aq, deg0c, deg1c, wpostt)


def _k6_body(h_ref, hw_ref, la_ref, lb_ref, bp_ref, res_ref,
             ia_v, ib_v, ga_v, gb_v, res_v, bp_v, sem):
    cid = lax.axis_index("c")
    sid = lax.axis_index("s")
    w = sid * NC + cid

    pltpu.sync_copy(la_ref.at[w], ia_v)
    pltpu.sync_copy(lb_ref.at[w], ib_v)
    pltpu.sync_copy(bp_ref, bp_v)
    c = jnp.sum(bp_v[...])
    i16 = lax.iota(jnp.int32, 16)

    for j in range(L_CHUNKS):
        pltpu.async_copy(h_ref.at[ia_v.at[j]], ga_v, sem).wait()
        pltpu.async_copy(hw_ref.at[ib_v.at[j]], gb_v, sem).wait()

        def grp(g, _):
            vec = jnp.zeros((16,), jnp.float32)
            for i in range(16):
                p = g * 16 + i
                acc = jnp.zeros((16,), jnp.float32)
                for k in range(16):
                    acc = acc + (ga_v[p, pl.ds(k * 16, 16)]
                                 * gb_v[p, pl.ds(k * 16, 16)])
                s = jnp.sum(acc)
                vec = jnp.where(i16 == i, s, vec)
            res_v[pl.ds(j * 128 + g * 16, 16)] = vec + c
            return 0
        lax.fori_loop(0, 8, grp, 0)

    pltpu.sync_copy(res_v, res_ref.at[w])


def _label_score(h, hw, la, lb, bp16):
    f = functools.partial(
        pl.kernel,
        out_type=jax.ShapeDtypeStruct((NW, L_PER_W), jnp.float32),
        mesh=_mesh(),
        scratch_types=[
            pltpu.VMEM((L_CHUNKS, 128), jnp.int32),
            pltpu.VMEM((L_CHUNKS, 128), jnp.int32),
            pltpu.VMEM((128, FEAT), jnp.float32),
            pltpu.VMEM((128, FEAT), jnp.float32),
            pltpu.VMEM((L_PER_W,), jnp.float32),
            pltpu.VMEM((16,), jnp.float32),
            pltpu.SemaphoreType.DMA,
        ],
        compiler_params=pltpu.CompilerParams(needs_layout_passes=False),
    )
    return f(_k6_body)(h, hw, la, lb, bp16)


def kernel(x, edge_index_rel0, edge_index_rel1, edge_label_index,
           p0, Wih0, Whh0, bih0, bhh0, W0,
           p1, Wih1, Whh1, bih1, bhh1, W1,
           Wpost, bpost):
    p128 = jnp.pad(jnp.stack([p0, p1], axis=1), ((0, 0), (0, 126)))
    p2 = jnp.stack([p0, p1], axis=1)

    npad = E_PAD - N_EDGES
    ar = lax.iota(jnp.int32, npad)
    sent_src = ar % 64
    sent_dst = N_NODES + (ar % 128)

    def prep_edges(ei):
        s = jnp.concatenate([ei[0], sent_src]).reshape(NS, E_CHUNKS, 128)
        d = jnp.concatenate([ei[1], sent_dst]).reshape(NS, E_CHUNKS, 128)
        return s, d

    src0p, dst0p = prep_edges(edge_index_rel0)
    src1p, dst1p = prep_edges(edge_index_rel1)

    lpad = L_PAD - N_LABEL
    lar = lax.iota(jnp.int32, lpad) % 64
    la = jnp.concatenate([edge_label_index[0], lar]).reshape(NW, L_CHUNKS, 128)
    lb = jnp.concatenate([edge_label_index[1], lar]).reshape(NW, L_CHUNKS, 128)

    wiht0, whht0 = Wih0.T, Whh0.T
    wiht1, whht1 = Wih1.T, Whh1.T
    wpostt = Wpost.T
    bp16 = jnp.pad(bpost, (0, 14))

    S = _scores(x, p128)
    spad = jnp.pad(S[:, :2], ((0, 240), (0, 0)),
                   constant_values=-jnp.inf)
    s0 = spad[:, 0].reshape(80, 128)
    s1 = spad[:, 1].reshape(80, 128)
    idx0, idx1, tv0, tv1 = _topk(s0, s1, p2)

    deg0, deg1, xt0, xt1 = _deg_and_gather(dst0p, dst1p, idx0, idx1, x)
    deg0c = deg0[:N_NODES].reshape(N_NODES, 1)
    deg1c = deg1[:N_NODES].reshape(N_NODES, 1)

    wm0, wm1 = _gru(xt0, tv0.reshape(256, 1), wiht0, whht0, bih0, bhh0, W0,
                    xt1, tv1.reshape(256, 1), wiht1, whht1, bih1, bhh1, W1)
    aq = _xws(x, wm0, wm1, deg0c, deg1c)

    oq = _message_passing(src0p, dst0p, src1p, dst1p, aq)

    h, hw = _combine([o[:N_NODES] for o in oq], aq, deg0c, deg1c, wpostt)

    res = _label_score(h, hw, la, lb, bp16)
    return res.reshape(L_PAD)[:N_LABEL]


# speedup vs baseline: 11.1041x; 1.1314x over previous
import functools

import jax
import jax.numpy as jnp
from jax import lax
from jax.experimental import pallas as pl
from jax.experimental.pallas import tpu as pltpu
from jax.experimental.pallas import tpu_sc as plsc

N_NODES = 10000
FEAT = 256
N_EDGES = 160000
N_LABEL = 20000

NC = 2
NS = 16
NW = NC * NS

E_CHUNK = 256
E_CHUNKS = 40
E_PER_TILE = E_CHUNKS * E_CHUNK
E_PAD = NS * E_PER_TILE
ACC_ROWS = 10240
DEG_PER_TILE = ACC_ROWS // NS

L_CHUNKS = 5
L_PER_W = L_CHUNKS * 128
L_PAD = NW * L_PER_W

def _mesh():
    return plsc.VectorSubcoreMesh(core_axis_name="c", subcore_axis_name="s")


def _k1_body(x_ref, p_ref, s_ref):
    s_ref[...] = jnp.dot(x_ref[...], p_ref[...],
                         preferred_element_type=jnp.float32)


def _scores(x, p128):
    return pl.pallas_call(
        _k1_body,
        out_shape=jax.ShapeDtypeStruct((N_NODES, 128), jnp.float32),
    )(x, p128)


def _k1b_body(s0_ref, s1_ref, p2_ref, idx0_ref, idx1_ref, tv0_ref, tv1_ref):
    s0 = s0_ref[...]
    s1 = s1_ref[...]
    pp = p2_ref[...]
    nrm0 = jnp.sqrt(jnp.sum(pp[:, 0:1] * pp[:, 0:1]))
    nrm1 = jnp.sqrt(jnp.sum(pp[:, 1:2] * pp[:, 1:2]))
    inv0 = 1.0 / (nrm0 + 1e-16)
    inv1 = 1.0 / (nrm1 + 1e-16)

    fi = (lax.broadcasted_iota(jnp.int32, (80, 128), 0) * 128
          + lax.broadcasted_iota(jnp.int32, (80, 128), 1))
    i256 = lax.iota(jnp.int32, 256)
    neg = jnp.float32(-jnp.inf)
    big = jnp.int32(1 << 30)

    def step(r, carry):
        c0, c1, va0, va1, ia0, ia1 = carry

        m0 = jnp.max(c0)
        ix0 = jnp.min(jnp.where(c0 == m0, fi, big))
        c0 = jnp.where(fi == ix0, neg, c0)
        va0 = jnp.where(i256 == r, m0, va0)
        ia0 = jnp.where(i256 == r, ix0, ia0)

        m1 = jnp.max(c1)
        ix1 = jnp.min(jnp.where(c1 == m1, fi, big))
        c1 = jnp.where(fi == ix1, neg, c1)
        va1 = jnp.where(i256 == r, m1, va1)
        ia1 = jnp.where(i256 == r, ix1, ia1)

        return c0, c1, va0, va1, ia0, ia1

    init = (s0, s1,
            jnp.zeros((256,), jnp.float32), jnp.zeros((256,), jnp.float32),
            jnp.zeros((256,), jnp.int32), jnp.zeros((256,), jnp.int32))
    _, _, va0, va1, ia0, ia1 = lax.fori_loop(0, 256, step, init)

    idx0_ref[...] = ia0
    idx1_ref[...] = ia1
    tv0_ref[...] = jnp.tanh(va0 * inv0)
    tv1_ref[...] = jnp.tanh(va1 * inv1)


def _topk(s0, s1, p2):
    return pl.pallas_call(
        _k1b_body,
        out_shape=(
            jax.ShapeDtypeStruct((256,), jnp.int32),
            jax.ShapeDtypeStruct((256,), jnp.int32),
            jax.ShapeDtypeStruct((256,), jnp.float32),
            jax.ShapeDtypeStruct((256,), jnp.float32),
        ),
    )(s0, s1, p2)


def _k2_body(dst0_ref, dst1_ref, idx0_ref, idx1_ref, x_ref,
             deg0_ref, deg1_ref, xt0_ref, xt1_ref,
             idx_v, ones_v, gidx_v, rows_v, zrow_v, deg_sh, sem):
    cid = lax.axis_index("c")
    sid = lax.axis_index("s")

    z16 = jnp.zeros((16,), jnp.float32)
    o16 = jnp.ones((16,), jnp.float32)

    def zb(k, _):
        zrow_v[pl.ds(k * 16, 16)] = z16
        return 0
    lax.fori_loop(0, DEG_PER_TILE // 16, zb, 0)

    def ob(k, _):
        ones_v[pl.ds(k * 16, 16)] = o16
        return 0
    lax.fori_loop(0, E_CHUNK // 16, ob, 0)

    pltpu.sync_copy(zrow_v, deg_sh.at[pl.ds(sid * DEG_PER_TILE, DEG_PER_TILE)])
    plsc.subcore_barrier()

    @pl.when(cid == 0)
    def _():
        pltpu.sync_copy(dst0_ref.at[sid], idx_v)

    @pl.when(cid == 1)
    def _():
        pltpu.sync_copy(dst1_ref.at[sid], idx_v)

    def add1(j, _):
        pltpu.sync_copy(
            ones_v, deg_sh.at[idx_v.at[pl.ds(j * E_CHUNK, E_CHUNK)]], add=True)
        return 0
    lax.fori_loop(0, E_CHUNKS, add1, 0)
    plsc.subcore_barrier()

    @pl.when(cid == 0)
    def _():
        pltpu.sync_copy(deg_sh.at[pl.ds(sid * DEG_PER_TILE, DEG_PER_TILE)],
                        deg0_ref.at[pl.ds(sid * DEG_PER_TILE, DEG_PER_TILE)])

    @pl.when(cid == 1)
    def _():
        pltpu.sync_copy(deg_sh.at[pl.ds(sid * DEG_PER_TILE, DEG_PER_TILE)],
                        deg1_ref.at[pl.ds(sid * DEG_PER_TILE, DEG_PER_TILE)])

    @pl.when(cid == 0)
    def _():
        pltpu.sync_copy(idx0_ref.at[pl.ds(sid * 16, 16)], gidx_v)
        pltpu.async_copy(x_ref.at[gidx_v], rows_v, sem).wait()
        pltpu.sync_copy(rows_v, xt0_ref.at[pl.ds(sid * 16, 16)])

    @pl.when(cid == 1)
    def _():
        pltpu.sync_copy(idx1_ref.at[pl.ds(sid * 16, 16)], gidx_v)
        pltpu.async_copy(x_ref.at[gidx_v], rows_v, sem).wait()
        pltpu.sync_copy(rows_v, xt1_ref.at[pl.ds(sid * 16, 16)])


def _deg_and_gather(dst0p, dst1p, idx0, idx1, x):
    f = functools.partial(
        pl.kernel,
        out_type=(
            jax.ShapeDtypeStruct((ACC_ROWS,), jnp.float32),
            jax.ShapeDtypeStruct((ACC_ROWS,), jnp.float32),
            jax.ShapeDtypeStruct((256, FEAT), jnp.float32),
            jax.ShapeDtypeStruct((256, FEAT), jnp.float32),
        ),
        mesh=_mesh(),
        scratch_types=[
            pltpu.VMEM((E_PER_TILE,), jnp.int32),
            pltpu.VMEM((E_CHUNK,), jnp.float32),
            pltpu.VMEM((16,), jnp.int32),
            pltpu.VMEM((16, FEAT), jnp.float32),
            pltpu.VMEM((DEG_PER_TILE,), jnp.float32),
            pltpu.VMEM_SHARED((ACC_ROWS,), jnp.float32),
            pltpu.SemaphoreType.DMA,
        ],
    )
    return f(_k2_body)(dst0p, dst1p, idx0, idx1, x)


def _gru_half(xt, tv, wiht, whht, bih, bhh, w0):
    xts = xt * tv
    gi = jnp.dot(xts, wiht, preferred_element_type=jnp.float32) + bih[None, :]
    gh = jnp.dot(w0, whht, preferred_element_type=jnp.float32) + bhh[None, :]
    i_r, i_z, i_n = gi[:, :256], gi[:, 256:512], gi[:, 512:]
    h_r, h_z, h_n = gh[:, :256], gh[:, 256:512], gh[:, 512:]
    r = 1.0 / (1.0 + jnp.exp(-(i_r + h_r)))
    z = 1.0 / (1.0 + jnp.exp(-(i_z + h_z)))
    n = jnp.tanh(i_n + r * h_n)
    return (1.0 - z) * n + z * w0


def _k3a_body(xt0_ref, tv0_ref, wiht0_ref, whht0_ref, bih0_ref, bhh0_ref,
              w00_ref, xt1_ref, tv1_ref, wiht1_ref, whht1_ref, bih1_ref,
              bhh1_ref, w01_ref, wm0_ref, wm1_ref):
    wm0_ref[...] = _gru_half(xt0_ref[...], tv0_ref[...], wiht0_ref[...],
                             whht0_ref[...], bih0_ref[...], bhh0_ref[...],
                             w00_ref[...])
    wm1_ref[...] = _gru_half(xt1_ref[...], tv1_ref[...], wiht1_ref[...],
                             whht1_ref[...], bih1_ref[...], bhh1_ref[...],
                             w01_ref[...])


def _gru(xt0, tv0, wiht0, whht0, bih0, bhh0, w00,
         xt1, tv1, wiht1, whht1, bih1, bhh1, w01):
    return pl.pallas_call(
        _k3a_body,
        out_shape=(
            jax.ShapeDtypeStruct((FEAT, FEAT), jnp.float32),
            jax.ShapeDtypeStruct((FEAT, FEAT), jnp.float32),
        ),
    )(xt0, tv0, wiht0, whht0, bih0, bhh0, w00,
      xt1, tv1, wiht1, whht1, bih1, bhh1, w01)


_RB = 2000
NQ = 4
QW = FEAT // NQ


def _k3b_body(x_ref, wm0_ref, wm1_ref, deg0_ref, deg1_ref, *a_refs):
    xb = x_ref[...]
    d0 = lax.rsqrt(deg0_ref[...] + 1.0)
    d1 = lax.rsqrt(deg1_ref[...] + 1.0)
    xs0 = jnp.dot(xb, wm0_ref[...], preferred_element_type=jnp.float32) * d0
    xs1 = jnp.dot(xb, wm1_ref[...], preferred_element_type=jnp.float32) * d1
    for q in range(NQ):
        a_refs[q][...] = xs0[:, q * QW:(q + 1) * QW]
        a_refs[NQ + q][...] = xs1[:, q * QW:(q + 1) * QW]


def _xws(x, wm0, wm1, deg0c, deg1c):
    nblk = N_NODES // _RB
    quarter = jax.ShapeDtypeStruct((N_NODES, QW), jnp.float32)
    return pl.pallas_call(
        _k3b_body,
        grid=(nblk,),
        in_specs=[
            pl.BlockSpec((_RB, FEAT), lambda i: (i, 0)),
            pl.BlockSpec((FEAT, FEAT), lambda i: (0, 0)),
            pl.BlockSpec((FEAT, FEAT), lambda i: (0, 0)),
            pl.BlockSpec((_RB, 1), lambda i: (i, 0)),
            pl.BlockSpec((_RB, 1), lambda i: (i, 0)),
        ],
        out_specs=(pl.BlockSpec((_RB, QW), lambda i: (i, 0)),) * (2 * NQ),
        out_shape=(quarter,) * (2 * NQ),
    )(x, wm0, wm1, deg0c, deg1c)


def _k4_body(src0_ref, dst0_ref, src1_ref, dst1_ref, *refs):
    a_refs = refs[:2 * NQ]
    o_refs = refs[2 * NQ:4 * NQ]
    sidx_v, didx_v, rows0_v, rows1_v, zbuf_v, acc_sh, sem0, sem1 = refs[4 * NQ:]
    cid = lax.axis_index("c")
    sid = lax.axis_index("s")

    z16 = jnp.zeros((16,), jnp.float32)

    def zb(i, _):
        for k in range(QW // 16):
            zbuf_v[i, pl.ds(k * 16, 16)] = z16
        return 0
    lax.fori_loop(0, 128, zb, 0)

    @pl.when(cid == 0)
    def _():
        pltpu.sync_copy(src0_ref.at[sid], sidx_v)
        pltpu.sync_copy(dst0_ref.at[sid], didx_v)

    @pl.when(cid == 1)
    def _():
        pltpu.sync_copy(src1_ref.at[sid], sidx_v)
        pltpu.sync_copy(dst1_ref.at[sid], didx_v)

    def do_quarter(a_ref, o_ref):
        for j in range(DEG_PER_TILE // 128):
            pltpu.sync_copy(
                zbuf_v,
                acc_sh.at[pl.ds(sid * DEG_PER_TILE + j * 128, 128)])
        plsc.subcore_barrier()

        def sl(v, j):
            return v.at[pl.ds(j * E_CHUNK, E_CHUNK)]

        pltpu.async_copy(a_ref.at[sl(sidx_v, 0)], rows0_v, sem0)

        def step(j, _):
            @pl.when(j % 2 == 0)
            def _():
                pltpu.make_async_copy(
                    a_ref.at[sl(sidx_v, j)], rows0_v, sem0).wait()

                @pl.when(j < E_CHUNKS - 1)
                def _():
                    pltpu.async_copy(
                        a_ref.at[sl(sidx_v, j + 1)], rows1_v, sem1)
                pltpu.sync_copy(rows0_v, acc_sh.at[sl(didx_v, j)], add=True)

            @pl.when(j % 2 == 1)
            def _():
                pltpu.make_async_copy(
                    a_ref.at[sl(sidx_v, j)], rows1_v, sem1).wait()

                @pl.when(j < E_CHUNKS - 1)
                def _():
                    pltpu.async_copy(
                        a_ref.at[sl(sidx_v, j + 1)], rows0_v, sem0)
                pltpu.sync_copy(rows1_v, acc_sh.at[sl(didx_v, j)], add=True)
            return 0
        lax.fori_loop(0, E_CHUNKS, step, 0)
        plsc.subcore_barrier()

        for j in range(DEG_PER_TILE // 128):
            s = pl.ds(sid * DEG_PER_TILE + j * 128, 128)
            pltpu.sync_copy(acc_sh.at[s], o_ref.at[s])
        plsc.subcore_barrier()

    @pl.when(cid == 0)
    def _():
        for q in range(NQ):
            do_quarter(a_refs[q], o_refs[q])

    @pl.when(cid == 1)
    def _():
        for q in range(NQ):
            do_quarter(a_refs[NQ + q], o_refs[NQ + q])


def _message_passing(src0p, dst0p, src1p, dst1p, aq):
    acc = jax.ShapeDtypeStruct((ACC_ROWS, QW), jnp.float32)
    f = functools.partial(
        pl.kernel,
        out_type=(acc,) * (2 * NQ),
        mesh=_mesh(),
        scratch_types=[
            pltpu.VMEM((E_PER_TILE,), jnp.int32),
            pltpu.VMEM((E_PER_TILE,), jnp.int32),
            pltpu.VMEM((E_CHUNK, QW), jnp.float32),
            pltpu.VMEM((E_CHUNK, QW), jnp.float32),
            pltpu.VMEM((128, QW), jnp.float32),
            pltpu.VMEM_SHARED((ACC_ROWS, QW), jnp.float32),
            pltpu.SemaphoreType.DMA,
            pltpu.SemaphoreType.DMA,
        ],
        compiler_params=pltpu.CompilerParams(use_tc_tiling_on_sc=False, needs_layout_passes=False),
    )
    return f(_k4_body)(src0p, dst0p, src1p, dst1p, *aq)


def _k5_body(*refs):
    o_refs = refs[:2 * NQ]
    a_refs = refs[2 * NQ:4 * NQ]
    deg0_ref, deg1_ref, wpt_ref, h_ref, hw_ref = refs[4 * NQ:]
    d0 = lax.rsqrt(deg0_ref[...] + 1.0)
    d1 = lax.rsqrt(deg1_ref[...] + 1.0)
    parts = []
    for q in range(NQ):
        parts.append(d0 * (o_refs[q][...] + a_refs[q][...])
                     + d1 * (o_refs[NQ + q][...] + a_refs[NQ + q][...]))
    h = jnp.maximum(jnp.concatenate(parts, axis=1), 0.0)
    wsum = wpt_ref[0:1, :] + wpt_ref[1:2, :]
    h_ref[...] = h
    hw_ref[...] = h * wsum


def _combine(oq, aq, deg0c, deg1c, wpostt):
    nblk = N_NODES // _RB
    ob = pl.BlockSpec((_RB, QW), lambda i: (i, 0))
    return pl.pallas_call(
        _k5_body,
        grid=(nblk,),
        in_specs=[ob] * (4 * NQ)
        + [pl.BlockSpec((_RB, 1), lambda i: (i, 0)),
           pl.BlockSpec((_RB, 1), lambda i: (i, 0)),
           pl.BlockSpec((2, FEAT), lambda i: (0, 0))],
        out_specs=(pl.BlockSpec((_RB, FEAT), lambda i: (i, 0)),
                   pl.BlockSpec((_RB, FEAT), lambda i: (i, 0))),
        out_shape=(jax.ShapeDtypeStruct((N_NODES, FEAT), jnp.float32),
                   jax.ShapeDtypeStruct((N_NODES, FEAT), jnp.float32)),
    )(*oq, *aq, deg0c, deg1c, wpostt)


def _k6_body(h_ref, hw_ref, la_ref, lb_ref, bp_ref, res_ref,
             ia_v, ib_v, ga_v, gb_v, res_v, bp_v, sem):
    cid = lax.axis_index("c")
    sid = lax.axis_index("s")
    w = sid * NC + cid

    pltpu.sync_copy(la_ref.at[w], ia_v)
    pltpu.sync_copy(lb_ref.at[w], ib_v)
    pltpu.sync_copy(bp_ref, bp_v)
    c = jnp.sum(bp_v[...])
    i16 = lax.iota(jnp.int32, 16)

    for j in range(L_CHUNKS):
        pltpu.async_copy(h_ref.at[ia_v.at[j]], ga_v, sem).wait()
        pltpu.async_copy(hw_ref.at[ib_v.at[j]], gb_v, sem).wait()

        def grp(g, _):
            vec = jnp.zeros((16,), jnp.float32)
            for i in range(16):
                p = g * 16 + i
                acc = jnp.zeros((16,), jnp.float32)
                for k in range(16):
                    acc = acc + (ga_v[p, pl.ds(k * 16, 16)]
                                 * gb_v[p, pl.ds(k * 16, 16)])
                s = jnp.sum(acc)
                vec = jnp.where(i16 == i, s, vec)
            res_v[pl.ds(j * 128 + g * 16, 16)] = vec + c
            return 0
        lax.fori_loop(0, 8, grp, 0)

    pltpu.sync_copy(res_v, res_ref.at[w])


def _label_score(h, hw, la, lb, bp16):
    f = functools.partial(
        pl.kernel,
        out_type=jax.ShapeDtypeStruct((NW, L_PER_W), jnp.float32),
        mesh=_mesh(),
        scratch_types=[
            pltpu.VMEM((L_CHUNKS, 128), jnp.int32),
            pltpu.VMEM((L_CHUNKS, 128), jnp.int32),
            pltpu.VMEM((128, FEAT), jnp.float32),
            pltpu.VMEM((128, FEAT), jnp.float32),
            pltpu.VMEM((L_PER_W,), jnp.float32),
            pltpu.VMEM((16,), jnp.float32),
            pltpu.SemaphoreType.DMA,
        ],
        compiler_params=pltpu.CompilerParams(needs_layout_passes=False),
    )
    return f(_k6_body)(h, hw, la, lb, bp16)


def kernel(x, edge_index_rel0, edge_index_rel1, edge_label_index,
           p0, Wih0, Whh0, bih0, bhh0, W0,
           p1, Wih1, Whh1, bih1, bhh1, W1,
           Wpost, bpost):
    p128 = jnp.pad(jnp.stack([p0, p1], axis=1), ((0, 0), (0, 126)))
    p2 = jnp.stack([p0, p1], axis=1)

    npad = E_PAD - N_EDGES
    ar = lax.iota(jnp.int32, npad)
    sent_src = ar % 64
    sent_dst = N_NODES + (ar % 128)

    def prep_edges(ei):
        s = jnp.concatenate([ei[0], sent_src]).reshape(NS, E_PER_TILE)
        d = jnp.concatenate([ei[1], sent_dst]).reshape(NS, E_PER_TILE)
        return s, d

    src0p, dst0p = prep_edges(edge_index_rel0)
    src1p, dst1p = prep_edges(edge_index_rel1)

    lpad = L_PAD - N_LABEL
    lar = lax.iota(jnp.int32, lpad) % 64
    la = jnp.concatenate([edge_label_index[0], lar]).reshape(NW, L_CHUNKS, 128)
    lb = jnp.concatenate([edge_label_index[1], lar]).reshape(NW, L_CHUNKS, 128)

    wiht0, whht0 = Wih0.T, Whh0.T
    wiht1, whht1 = Wih1.T, Whh1.T
    wpostt = Wpost.T
    bp16 = jnp.pad(bpost, (0, 14))

    S = _scores(x, p128)
    spad = jnp.pad(S[:, :2], ((0, 240), (0, 0)),
                   constant_values=-jnp.inf)
    s0 = spad[:, 0].reshape(80, 128)
    s1 = spad[:, 1].reshape(80, 128)
    idx0, idx1, tv0, tv1 = _topk(s0, s1, p2)

    deg0, deg1, xt0, xt1 = _deg_and_gather(dst0p, dst1p, idx0, idx1, x)
    deg0c = deg0[:N_NODES].reshape(N_NODES, 1)
    deg1c = deg1[:N_NODES].reshape(N_NODES, 1)

    wm0, wm1 = _gru(xt0, tv0.reshape(256, 1), wiht0, whht0, bih0, bhh0, W0,
                    xt1, tv1.reshape(256, 1), wiht1, whht1, bih1, bhh1, W1)
    aq = _xws(x, wm0, wm1, deg0c, deg1c)

    oq = _message_passing(src0p, dst0p, src1p, dst1p, aq)

    h, hw = _combine([o[:N_NODES] for o in oq], aq, deg0c, deg1c, wpostt)

    res = _label_score(h, hw, la, lb, bp16)
    return res.reshape(L_PAD)[:N_LABEL]


# speedup vs baseline: 11.5221x; 1.0376x over previous
import functools

import jax
import jax.numpy as jnp
from jax import lax
from jax.experimental import pallas as pl
from jax.experimental.pallas import tpu as pltpu
from jax.experimental.pallas import tpu_sc as plsc

N_NODES = 10000
FEAT = 256
N_EDGES = 160000
N_LABEL = 20000

NC = 2
NS = 16
NW = NC * NS

E_CHUNK = 512
E_CHUNKS = 20
E_PER_TILE = E_CHUNKS * E_CHUNK
E_PAD = NS * E_PER_TILE
ACC_ROWS = 10240
DEG_PER_TILE = ACC_ROWS // NS

L_CHUNKS = 5
L_PER_W = L_CHUNKS * 128
L_PAD = NW * L_PER_W

def _mesh():
    return plsc.VectorSubcoreMesh(core_axis_name="c", subcore_axis_name="s")


def _k1_body(x_ref, p_ref, s_ref):
    s_ref[...] = jnp.dot(x_ref[...], p_ref[...],
                         preferred_element_type=jnp.float32)


def _scores(x, p128):
    return pl.pallas_call(
        _k1_body,
        out_shape=jax.ShapeDtypeStruct((N_NODES, 128), jnp.float32),
    )(x, p128)


def _k1b_body(s0_ref, s1_ref, p2_ref, idx0_ref, idx1_ref, tv0_ref, tv1_ref):
    s0 = s0_ref[...]
    s1 = s1_ref[...]
    pp = p2_ref[...]
    nrm0 = jnp.sqrt(jnp.sum(pp[:, 0:1] * pp[:, 0:1]))
    nrm1 = jnp.sqrt(jnp.sum(pp[:, 1:2] * pp[:, 1:2]))
    inv0 = 1.0 / (nrm0 + 1e-16)
    inv1 = 1.0 / (nrm1 + 1e-16)

    fi = (lax.broadcasted_iota(jnp.int32, (80, 128), 0) * 128
          + lax.broadcasted_iota(jnp.int32, (80, 128), 1))
    i256 = lax.iota(jnp.int32, 256)
    neg = jnp.float32(-jnp.inf)
    big = jnp.int32(1 << 30)

    def step(r, carry):
        c0, c1, va0, va1, ia0, ia1 = carry

        m0 = jnp.max(c0)
        ix0 = jnp.min(jnp.where(c0 == m0, fi, big))
        c0 = jnp.where(fi == ix0, neg, c0)
        va0 = jnp.where(i256 == r, m0, va0)
        ia0 = jnp.where(i256 == r, ix0, ia0)

        m1 = jnp.max(c1)
        ix1 = jnp.min(jnp.where(c1 == m1, fi, big))
        c1 = jnp.where(fi == ix1, neg, c1)
        va1 = jnp.where(i256 == r, m1, va1)
        ia1 = jnp.where(i256 == r, ix1, ia1)

        return c0, c1, va0, va1, ia0, ia1

    init = (s0, s1,
            jnp.zeros((256,), jnp.float32), jnp.zeros((256,), jnp.float32),
            jnp.zeros((256,), jnp.int32), jnp.zeros((256,), jnp.int32))
    _, _, va0, va1, ia0, ia1 = lax.fori_loop(0, 256, step, init)

    idx0_ref[...] = ia0
    idx1_ref[...] = ia1
    tv0_ref[...] = jnp.tanh(va0 * inv0)
    tv1_ref[...] = jnp.tanh(va1 * inv1)


def _topk(s0, s1, p2):
    return pl.pallas_call(
        _k1b_body,
        out_shape=(
            jax.ShapeDtypeStruct((256,), jnp.int32),
            jax.ShapeDtypeStruct((256,), jnp.int32),
            jax.ShapeDtypeStruct((256,), jnp.float32),
            jax.ShapeDtypeStruct((256,), jnp.float32),
        ),
    )(s0, s1, p2)


def _k2_body(dst0_ref, dst1_ref, idx0_ref, idx1_ref, x_ref,
             deg0_ref, deg1_ref, xt0_ref, xt1_ref,
             idx_v, ones_v, gidx_v, rows_v, zrow_v, deg_sh, sem):
    cid = lax.axis_index("c")
    sid = lax.axis_index("s")

    z16 = jnp.zeros((16,), jnp.float32)
    o16 = jnp.ones((16,), jnp.float32)

    def zb(k, _):
        zrow_v[pl.ds(k * 16, 16)] = z16
        return 0
    lax.fori_loop(0, DEG_PER_TILE // 16, zb, 0)

    def ob(k, _):
        ones_v[pl.ds(k * 16, 16)] = o16
        return 0
    lax.fori_loop(0, E_CHUNK // 16, ob, 0)

    pltpu.sync_copy(zrow_v, deg_sh.at[pl.ds(sid * DEG_PER_TILE, DEG_PER_TILE)])
    plsc.subcore_barrier()

    @pl.when(cid == 0)
    def _():
        pltpu.sync_copy(dst0_ref.at[sid], idx_v)

    @pl.when(cid == 1)
    def _():
        pltpu.sync_copy(dst1_ref.at[sid], idx_v)

    def add1(j, _):
        pltpu.sync_copy(
            ones_v, deg_sh.at[idx_v.at[pl.ds(j * E_CHUNK, E_CHUNK)]], add=True)
        return 0
    lax.fori_loop(0, E_CHUNKS, add1, 0)
    plsc.subcore_barrier()

    @pl.when(cid == 0)
    def _():
        pltpu.sync_copy(deg_sh.at[pl.ds(sid * DEG_PER_TILE, DEG_PER_TILE)],
                        deg0_ref.at[pl.ds(sid * DEG_PER_TILE, DEG_PER_TILE)])

    @pl.when(cid == 1)
    def _():
        pltpu.sync_copy(deg_sh.at[pl.ds(sid * DEG_PER_TILE, DEG_PER_TILE)],
                        deg1_ref.at[pl.ds(sid * DEG_PER_TILE, DEG_PER_TILE)])

    @pl.when(cid == 0)
    def _():
        pltpu.sync_copy(idx0_ref.at[pl.ds(sid * 16, 16)], gidx_v)
        pltpu.async_copy(x_ref.at[gidx_v], rows_v, sem).wait()
        pltpu.sync_copy(rows_v, xt0_ref.at[pl.ds(sid * 16, 16)])

    @pl.when(cid == 1)
    def _():
        pltpu.sync_copy(idx1_ref.at[pl.ds(sid * 16, 16)], gidx_v)
        pltpu.async_copy(x_ref.at[gidx_v], rows_v, sem).wait()
        pltpu.sync_copy(rows_v, xt1_ref.at[pl.ds(sid * 16, 16)])


def _deg_and_gather(dst0p, dst1p, idx0, idx1, x):
    f = functools.partial(
        pl.kernel,
        out_type=(
            jax.ShapeDtypeStruct((ACC_ROWS,), jnp.float32),
            jax.ShapeDtypeStruct((ACC_ROWS,), jnp.float32),
            jax.ShapeDtypeStruct((256, FEAT), jnp.float32),
            jax.ShapeDtypeStruct((256, FEAT), jnp.float32),
        ),
        mesh=_mesh(),
        scratch_types=[
            pltpu.VMEM((E_PER_TILE,), jnp.int32),
            pltpu.VMEM((E_CHUNK,), jnp.float32),
            pltpu.VMEM((16,), jnp.int32),
            pltpu.VMEM((16, FEAT), jnp.float32),
            pltpu.VMEM((DEG_PER_TILE,), jnp.float32),
            pltpu.VMEM_SHARED((ACC_ROWS,), jnp.float32),
            pltpu.SemaphoreType.DMA,
        ],
    )
    return f(_k2_body)(dst0p, dst1p, idx0, idx1, x)


def _gru_half(xt, tv, wiht, whht, bih, bhh, w0):
    xts = xt * tv
    gi = jnp.dot(xts, wiht, preferred_element_type=jnp.float32) + bih[None, :]
    gh = jnp.dot(w0, whht, preferred_element_type=jnp.float32) + bhh[None, :]
    i_r, i_z, i_n = gi[:, :256], gi[:, 256:512], gi[:, 512:]
    h_r, h_z, h_n = gh[:, :256], gh[:, 256:512], gh[:, 512:]
    r = 1.0 / (1.0 + jnp.exp(-(i_r + h_r)))
    z = 1.0 / (1.0 + jnp.exp(-(i_z + h_z)))
    n = jnp.tanh(i_n + r * h_n)
    return (1.0 - z) * n + z * w0


def _k3a_body(xt0_ref, tv0_ref, wiht0_ref, whht0_ref, bih0_ref, bhh0_ref,
              w00_ref, xt1_ref, tv1_ref, wiht1_ref, whht1_ref, bih1_ref,
              bhh1_ref, w01_ref, wm0_ref, wm1_ref):
    wm0_ref[...] = _gru_half(xt0_ref[...], tv0_ref[...], wiht0_ref[...],
                             whht0_ref[...], bih0_ref[...], bhh0_ref[...],
                             w00_ref[...])
    wm1_ref[...] = _gru_half(xt1_ref[...], tv1_ref[...], wiht1_ref[...],
                             whht1_ref[...], bih1_ref[...], bhh1_ref[...],
                             w01_ref[...])


def _gru(xt0, tv0, wiht0, whht0, bih0, bhh0, w00,
         xt1, tv1, wiht1, whht1, bih1, bhh1, w01):
    return pl.pallas_call(
        _k3a_body,
        out_shape=(
            jax.ShapeDtypeStruct((FEAT, FEAT), jnp.float32),
            jax.ShapeDtypeStruct((FEAT, FEAT), jnp.float32),
        ),
    )(xt0, tv0, wiht0, whht0, bih0, bhh0, w00,
      xt1, tv1, wiht1, whht1, bih1, bhh1, w01)


_RB = 2000
NQ = 4
QW = FEAT // NQ


def _k3b_body(x_ref, wm0_ref, wm1_ref, deg0_ref, deg1_ref, *a_refs):
    xb = x_ref[...]
    d0 = lax.rsqrt(deg0_ref[...] + 1.0)
    d1 = lax.rsqrt(deg1_ref[...] + 1.0)
    xs0 = jnp.dot(xb, wm0_ref[...], preferred_element_type=jnp.float32) * d0
    xs1 = jnp.dot(xb, wm1_ref[...], preferred_element_type=jnp.float32) * d1
    for q in range(NQ):
        a_refs[q][...] = xs0[:, q * QW:(q + 1) * QW]
        a_refs[NQ + q][...] = xs1[:, q * QW:(q + 1) * QW]


def _xws(x, wm0, wm1, deg0c, deg1c):
    nblk = N_NODES // _RB
    quarter = jax.ShapeDtypeStruct((N_NODES, QW), jnp.float32)
    return pl.pallas_call(
        _k3b_body,
        grid=(nblk,),
        in_specs=[
            pl.BlockSpec((_RB, FEAT), lambda i: (i, 0)),
            pl.BlockSpec((FEAT, FEAT), lambda i: (0, 0)),
            pl.BlockSpec((FEAT, FEAT), lambda i: (0, 0)),
            pl.BlockSpec((_RB, 1), lambda i: (i, 0)),
            pl.BlockSpec((_RB, 1), lambda i: (i, 0)),
        ],
        out_specs=(pl.BlockSpec((_RB, QW), lambda i: (i, 0)),) * (2 * NQ),
        out_shape=(quarter,) * (2 * NQ),
    )(x, wm0, wm1, deg0c, deg1c)


def _k4_body(src0_ref, dst0_ref, src1_ref, dst1_ref, *refs):
    a_refs = refs[:2 * NQ]
    o_refs = refs[2 * NQ:4 * NQ]
    sidx_v, didx_v, rows0_v, rows1_v, zbuf_v, acc_sh, sem0, sem1 = refs[4 * NQ:]
    cid = lax.axis_index("c")
    sid = lax.axis_index("s")

    z16 = jnp.zeros((16,), jnp.float32)

    def zb(i, _):
        for k in range(QW // 16):
            zbuf_v[i, pl.ds(k * 16, 16)] = z16
        return 0
    lax.fori_loop(0, 64, zb, 0)

    @pl.when(cid == 0)
    def _():
        pltpu.sync_copy(src0_ref.at[sid], sidx_v)
        pltpu.sync_copy(dst0_ref.at[sid], didx_v)

    @pl.when(cid == 1)
    def _():
        pltpu.sync_copy(src1_ref.at[sid], sidx_v)
        pltpu.sync_copy(dst1_ref.at[sid], didx_v)

    def do_quarter(a_ref, o_ref):
        for j in range(DEG_PER_TILE // 64):
            pltpu.sync_copy(
                zbuf_v,
                acc_sh.at[pl.ds(sid * DEG_PER_TILE + j * 64, 64)])
        plsc.subcore_barrier()

        def sl(v, j):
            return v.at[pl.ds(j * E_CHUNK, E_CHUNK)]

        pltpu.async_copy(a_ref.at[sl(sidx_v, 0)], rows0_v, sem0)

        def step(j, _):
            @pl.when(j % 2 == 0)
            def _():
                pltpu.make_async_copy(
                    a_ref.at[sl(sidx_v, j)], rows0_v, sem0).wait()

                @pl.when(j < E_CHUNKS - 1)
                def _():
                    pltpu.async_copy(
                        a_ref.at[sl(sidx_v, j + 1)], rows1_v, sem1)
                pltpu.sync_copy(rows0_v, acc_sh.at[sl(didx_v, j)], add=True)

            @pl.when(j % 2 == 1)
            def _():
                pltpu.make_async_copy(
                    a_ref.at[sl(sidx_v, j)], rows1_v, sem1).wait()

                @pl.when(j < E_CHUNKS - 1)
                def _():
                    pltpu.async_copy(
                        a_ref.at[sl(sidx_v, j + 1)], rows0_v, sem0)
                pltpu.sync_copy(rows1_v, acc_sh.at[sl(didx_v, j)], add=True)
            return 0
        lax.fori_loop(0, E_CHUNKS, step, 0)
        plsc.subcore_barrier()

        for j in range(DEG_PER_TILE // 128):
            s = pl.ds(sid * DEG_PER_TILE + j * 128, 128)
            pltpu.sync_copy(acc_sh.at[s], o_ref.at[s])
        plsc.subcore_barrier()

    @pl.when(cid == 0)
    def _():
        for q in range(NQ):
            do_quarter(a_refs[q], o_refs[q])

    @pl.when(cid == 1)
    def _():
        for q in range(NQ):
            do_quarter(a_refs[NQ + q], o_refs[NQ + q])


def _message_passing(src0p, dst0p, src1p, dst1p, aq):
    acc = jax.ShapeDtypeStruct((ACC_ROWS, QW), jnp.float32)
    f = functools.partial(
        pl.kernel,
        out_type=(acc,) * (2 * NQ),
        mesh=_mesh(),
        scratch_types=[
            pltpu.VMEM((E_PER_TILE,), jnp.int32),
            pltpu.VMEM((E_PER_TILE,), jnp.int32),
            pltpu.VMEM((E_CHUNK, QW), jnp.float32),
            pltpu.VMEM((E_CHUNK, QW), jnp.float32),
            pltpu.VMEM((64, QW), jnp.float32),
            pltpu.VMEM_SHARED((ACC_ROWS, QW), jnp.float32),
            pltpu.SemaphoreType.DMA,
            pltpu.SemaphoreType.DMA,
        ],
        compiler_params=pltpu.CompilerParams(use_tc_tiling_on_sc=False, needs_layout_passes=False),
    )
    return f(_k4_body)(src0p, dst0p, src1p, dst1p, *aq)


def _k5_body(*refs):
    o_refs = refs[:2 * NQ]
    a_refs = refs[2 * NQ:4 * NQ]
    deg0_ref, deg1_ref, wpt_ref, h_ref, hw_ref = refs[4 * NQ:]
    d0 = lax.rsqrt(deg0_ref[...] + 1.0)
    d1 = lax.rsqrt(deg1_ref[...] + 1.0)
    parts = []
    for q in range(NQ):
        parts.append(d0 * (o_refs[q][...] + a_refs[q][...])
                     + d1 * (o_refs[NQ + q][...] + a_refs[NQ + q][...]))
    h = jnp.maximum(jnp.concatenate(parts, axis=1), 0.0)
    wsum = wpt_ref[0:1, :] + wpt_ref[1:2, :]
    h_ref[...] = h
    hw_ref[...] = h * wsum


def _combine(oq, aq, deg0c, deg1c, wpostt):
    nblk = N_NODES // _RB
    ob = pl.BlockSpec((_RB, QW), lambda i: (i, 0))
    return pl.pallas_call(
        _k5_body,
        grid=(nblk,),
        in_specs=[ob] * (4 * NQ)
        + [pl.BlockSpec((_RB, 1), lambda i: (i, 0)),
           pl.BlockSpec((_RB, 1), lambda i: (i, 0)),
           pl.BlockSpec((2, FEAT), lambda i: (0, 0))],
        out_specs=(pl.BlockSpec((_RB, FEAT), lambda i: (i, 0)),
                   pl.BlockSpec((_RB, FEAT), lambda i: (i, 0))),
        out_shape=(jax.ShapeDtypeStruct((N_NODES, FEAT), jnp.float32),
                   jax.ShapeDtypeStruct((N_NODES, FEAT), jnp.float32)),
    )(*oq, *aq, deg0c, deg1c, wpostt)


def _k6_body(h_ref, hw_ref, la_ref, lb_ref, bp_ref, res_ref,
             ia_v, ib_v, ga_v, gb_v, res_v, bp_v, sem):
    cid = lax.axis_index("c")
    sid = lax.axis_index("s")
    w = sid * NC + cid

    pltpu.sync_copy(la_ref.at[w], ia_v)
    pltpu.sync_copy(lb_ref.at[w], ib_v)
    pltpu.sync_copy(bp_ref, bp_v)
    c = jnp.sum(bp_v[...])
    i16 = lax.iota(jnp.int32, 16)

    for j in range(L_CHUNKS):
        pltpu.async_copy(h_ref.at[ia_v.at[j]], ga_v, sem).wait()
        pltpu.async_copy(hw_ref.at[ib_v.at[j]], gb_v, sem).wait()

        def grp(g, _):
            vec = jnp.zeros((16,), jnp.float32)
            for i in range(16):
                p = g * 16 + i
                acc = jnp.zeros((16,), jnp.float32)
                for k in range(16):
                    acc = acc + (ga_v[p, pl.ds(k * 16, 16)]
                                 * gb_v[p, pl.ds(k * 16, 16)])
                s = jnp.sum(acc)
                vec = jnp.where(i16 == i, s, vec)
            res_v[pl.ds(j * 128 + g * 16, 16)] = vec + c
            return 0
        lax.fori_loop(0, 8, grp, 0)

    pltpu.sync_copy(res_v, res_ref.at[w])


def _label_score(h, hw, la, lb, bp16):
    f = functools.partial(
        pl.kernel,
        out_type=jax.ShapeDtypeStruct((NW, L_PER_W), jnp.float32),
        mesh=_mesh(),
        scratch_types=[
            pltpu.VMEM((L_CHUNKS, 128), jnp.int32),
            pltpu.VMEM((L_CHUNKS, 128), jnp.int32),
            pltpu.VMEM((128, FEAT), jnp.float32),
            pltpu.VMEM((128, FEAT), jnp.float32),
            pltpu.VMEM((L_PER_W,), jnp.float32),
            pltpu.VMEM((16,), jnp.float32),
            pltpu.SemaphoreType.DMA,
        ],
        compiler_params=pltpu.CompilerParams(needs_layout_passes=False),
    )
    return f(_k6_body)(h, hw, la, lb, bp16)


def kernel(x, edge_index_rel0, edge_index_rel1, edge_label_index,
           p0, Wih0, Whh0, bih0, bhh0, W0,
           p1, Wih1, Whh1, bih1, bhh1, W1,
           Wpost, bpost):
    p128 = jnp.pad(jnp.stack([p0, p1], axis=1), ((0, 0), (0, 126)))
    p2 = jnp.stack([p0, p1], axis=1)

    npad = E_PAD - N_EDGES
    ar = lax.iota(jnp.int32, npad)
    sent_src = ar % 64
    sent_dst = N_NODES + (ar % 128)

    def prep_edges(ei):
        s = jnp.concatenate([ei[0], sent_src]).reshape(NS, E_PER_TILE)
        d = jnp.concatenate([ei[1], sent_dst]).reshape(NS, E_PER_TILE)
        return s, d

    src0p, dst0p = prep_edges(edge_index_rel0)
    src1p, dst1p = prep_edges(edge_index_rel1)

    lpad = L_PAD - N_LABEL
    lar = lax.iota(jnp.int32, lpad) % 64
    la = jnp.concatenate([edge_label_index[0], lar]).reshape(NW, L_CHUNKS, 128)
    lb = jnp.concatenate([edge_label_index[1], lar]).reshape(NW, L_CHUNKS, 128)

    wiht0, whht0 = Wih0.T, Whh0.T
    wiht1, whht1 = Wih1.T, Whh1.T
    wpostt = Wpost.T
    bp16 = jnp.pad(bpost, (0, 14))

    S = _scores(x, p128)
    spad = jnp.pad(S[:, :2], ((0, 240), (0, 0)),
                   constant_values=-jnp.inf)
    s0 = spad[:, 0].reshape(80, 128)
    s1 = spad[:, 1].reshape(80, 128)
    idx0, idx1, tv0, tv1 = _topk(s0, s1, p2)

    deg0, deg1, xt0, xt1 = _deg_and_gather(dst0p, dst1p, idx0, idx1, x)
    deg0c = deg0[:N_NODES].reshape(N_NODES, 1)
    deg1c = deg1[:N_NODES].reshape(N_NODES, 1)

    wm0, wm1 = _gru(xt0, tv0.reshape(256, 1), wiht0, whht0, bih0, bhh0, W0,
                    xt1, tv1.reshape(256, 1), wiht1, whht1, bih1, bhh1, W1)
    aq = _xws(x, wm0, wm1, deg0c, deg1c)

    oq = _message_passing(src0p, dst0p, src1p, dst1p, aq)

    h, hw = _combine([o[:N_NODES] for o in oq], aq, deg0c, deg1c, wpostt)

    res = _label_score(h, hw, la, lb, bp16)
    return res.reshape(L_PAD)[:N_LABEL]


# speedup vs baseline: 11.8425x; 1.0278x over previous
import functools

import jax
import jax.numpy as jnp
from jax import lax
from jax.experimental import pallas as pl
from jax.experimental.pallas import tpu as pltpu
from jax.experimental.pallas import tpu_sc as plsc

N_NODES = 10000
FEAT = 256
N_EDGES = 160000
N_LABEL = 20000

NC = 2
NS = 16
NW = NC * NS

E_CHUNK = 512
E_CHUNKS = 20
E_PER_TILE = E_CHUNKS * E_CHUNK
E_PAD = NS * E_PER_TILE
ACC_ROWS = 10240
DEG_PER_TILE = ACC_ROWS // NS

L_CHUNKS = 5
L_PER_W = L_CHUNKS * 128
L_PAD = NW * L_PER_W

def _mesh():
    return plsc.VectorSubcoreMesh(core_axis_name="c", subcore_axis_name="s")


def _k1_body(x_ref, p_ref, s_ref):
    s_ref[...] = jnp.dot(x_ref[...], p_ref[...],
                         preferred_element_type=jnp.float32)


def _scores(x, p128):
    return pl.pallas_call(
        _k1_body,
        out_shape=jax.ShapeDtypeStruct((N_NODES, 128), jnp.float32),
    )(x, p128)


def _k1b_body(s0_ref, s1_ref, p2_ref, idx0_ref, idx1_ref, tv0_ref, tv1_ref):
    s0 = s0_ref[...]
    s1 = s1_ref[...]
    pp = p2_ref[...]
    nrm0 = jnp.sqrt(jnp.sum(pp[:, 0:1] * pp[:, 0:1]))
    nrm1 = jnp.sqrt(jnp.sum(pp[:, 1:2] * pp[:, 1:2]))
    inv0 = 1.0 / (nrm0 + 1e-16)
    inv1 = 1.0 / (nrm1 + 1e-16)

    fi = (lax.broadcasted_iota(jnp.int32, (80, 128), 0) * 128
          + lax.broadcasted_iota(jnp.int32, (80, 128), 1))
    i256 = lax.iota(jnp.int32, 256)
    neg = jnp.float32(-jnp.inf)
    big = jnp.int32(1 << 30)

    def step(r, carry):
        c0, c1, va0, va1, ia0, ia1 = carry

        m0 = jnp.max(c0)
        ix0 = jnp.min(jnp.where(c0 == m0, fi, big))
        c0 = jnp.where(fi == ix0, neg, c0)
        va0 = jnp.where(i256 == r, m0, va0)
        ia0 = jnp.where(i256 == r, ix0, ia0)

        m1 = jnp.max(c1)
        ix1 = jnp.min(jnp.where(c1 == m1, fi, big))
        c1 = jnp.where(fi == ix1, neg, c1)
        va1 = jnp.where(i256 == r, m1, va1)
        ia1 = jnp.where(i256 == r, ix1, ia1)

        return c0, c1, va0, va1, ia0, ia1

    init = (s0, s1,
            jnp.zeros((256,), jnp.float32), jnp.zeros((256,), jnp.float32),
            jnp.zeros((256,), jnp.int32), jnp.zeros((256,), jnp.int32))
    _, _, va0, va1, ia0, ia1 = lax.fori_loop(0, 256, step, init)

    idx0_ref[...] = ia0
    idx1_ref[...] = ia1
    tv0_ref[...] = jnp.tanh(va0 * inv0)
    tv1_ref[...] = jnp.tanh(va1 * inv1)


def _topk(s0, s1, p2):
    return pl.pallas_call(
        _k1b_body,
        out_shape=(
            jax.ShapeDtypeStruct((256,), jnp.int32),
            jax.ShapeDtypeStruct((256,), jnp.int32),
            jax.ShapeDtypeStruct((256,), jnp.float32),
            jax.ShapeDtypeStruct((256,), jnp.float32),
        ),
    )(s0, s1, p2)


def _k2_body(dst0_ref, dst1_ref, idx0_ref, idx1_ref, x_ref,
             deg0_ref, deg1_ref, xt0_ref, xt1_ref,
             idx_v, ones_v, gidx_v, rows_v, zrow_v, deg_sh, sem):
    cid = lax.axis_index("c")
    sid = lax.axis_index("s")

    z16 = jnp.zeros((16,), jnp.float32)
    o16 = jnp.ones((16,), jnp.float32)

    def zb(k, _):
        zrow_v[pl.ds(k * 16, 16)] = z16
        return 0
    lax.fori_loop(0, DEG_PER_TILE // 16, zb, 0)

    def ob(k, _):
        ones_v[pl.ds(k * 16, 16)] = o16
        return 0
    lax.fori_loop(0, E_CHUNK // 16, ob, 0)

    pltpu.sync_copy(zrow_v, deg_sh.at[pl.ds(sid * DEG_PER_TILE, DEG_PER_TILE)])
    plsc.subcore_barrier()

    @pl.when(cid == 0)
    def _():
        pltpu.sync_copy(dst0_ref.at[sid], idx_v)

    @pl.when(cid == 1)
    def _():
        pltpu.sync_copy(dst1_ref.at[sid], idx_v)

    def add1(j, _):
        pltpu.sync_copy(
            ones_v, deg_sh.at[idx_v.at[pl.ds(j * E_CHUNK, E_CHUNK)]], add=True)
        return 0
    lax.fori_loop(0, E_CHUNKS, add1, 0)
    plsc.subcore_barrier()

    @pl.when(cid == 0)
    def _():
        pltpu.sync_copy(deg_sh.at[pl.ds(sid * DEG_PER_TILE, DEG_PER_TILE)],
                        deg0_ref.at[pl.ds(sid * DEG_PER_TILE, DEG_PER_TILE)])

    @pl.when(cid == 1)
    def _():
        pltpu.sync_copy(deg_sh.at[pl.ds(sid * DEG_PER_TILE, DEG_PER_TILE)],
                        deg1_ref.at[pl.ds(sid * DEG_PER_TILE, DEG_PER_TILE)])

    @pl.when(cid == 0)
    def _():
        pltpu.sync_copy(idx0_ref.at[pl.ds(sid * 16, 16)], gidx_v)
        pltpu.async_copy(x_ref.at[gidx_v], rows_v, sem).wait()
        pltpu.sync_copy(rows_v, xt0_ref.at[pl.ds(sid * 16, 16)])

    @pl.when(cid == 1)
    def _():
        pltpu.sync_copy(idx1_ref.at[pl.ds(sid * 16, 16)], gidx_v)
        pltpu.async_copy(x_ref.at[gidx_v], rows_v, sem).wait()
        pltpu.sync_copy(rows_v, xt1_ref.at[pl.ds(sid * 16, 16)])


def _deg_and_gather(dst0p, dst1p, idx0, idx1, x):
    f = functools.partial(
        pl.kernel,
        out_type=(
            jax.ShapeDtypeStruct((ACC_ROWS,), jnp.float32),
            jax.ShapeDtypeStruct((ACC_ROWS,), jnp.float32),
            jax.ShapeDtypeStruct((256, FEAT), jnp.float32),
            jax.ShapeDtypeStruct((256, FEAT), jnp.float32),
        ),
        mesh=_mesh(),
        scratch_types=[
            pltpu.VMEM((E_PER_TILE,), jnp.int32),
            pltpu.VMEM((E_CHUNK,), jnp.float32),
            pltpu.VMEM((16,), jnp.int32),
            pltpu.VMEM((16, FEAT), jnp.float32),
            pltpu.VMEM((DEG_PER_TILE,), jnp.float32),
            pltpu.VMEM_SHARED((ACC_ROWS,), jnp.float32),
            pltpu.SemaphoreType.DMA,
        ],
    )
    return f(_k2_body)(dst0p, dst1p, idx0, idx1, x)


def _gru_half(xt, tv, wiht, whht, bih, bhh, w0):
    xts = xt * tv
    gi = jnp.dot(xts, wiht, preferred_element_type=jnp.float32) + bih[None, :]
    gh = jnp.dot(w0, whht, preferred_element_type=jnp.float32) + bhh[None, :]
    i_r, i_z, i_n = gi[:, :256], gi[:, 256:512], gi[:, 512:]
    h_r, h_z, h_n = gh[:, :256], gh[:, 256:512], gh[:, 512:]
    r = 1.0 / (1.0 + jnp.exp(-(i_r + h_r)))
    z = 1.0 / (1.0 + jnp.exp(-(i_z + h_z)))
    n = jnp.tanh(i_n + r * h_n)
    return (1.0 - z) * n + z * w0


_RB = ACC_ROWS // 8
NQ = 4
QW = FEAT // NQ


def _k3_body(xt0_ref, tv0_ref, wiht0_ref, whht0_ref, bih0_ref, bhh0_ref,
             w00_ref, xt1_ref, tv1_ref, wiht1_ref, whht1_ref, bih1_ref,
             bhh1_ref, w01_ref, x_ref, deg0_ref, deg1_ref, *rest):
    a_refs = rest[:2 * NQ]
    wm0_s, wm1_s = rest[2 * NQ:]
    @pl.when(pl.program_id(0) == 0)
    def _():
        wm0_s[...] = _gru_half(xt0_ref[...], tv0_ref[...], wiht0_ref[...],
                               whht0_ref[...], bih0_ref[...], bhh0_ref[...],
                               w00_ref[...])
        wm1_s[...] = _gru_half(xt1_ref[...], tv1_ref[...], wiht1_ref[...],
                               whht1_ref[...], bih1_ref[...], bhh1_ref[...],
                               w01_ref[...])
    xb = x_ref[...]
    d0 = lax.rsqrt(deg0_ref[...] + 1.0)
    d1 = lax.rsqrt(deg1_ref[...] + 1.0)
    xs0 = jnp.dot(xb, wm0_s[...], preferred_element_type=jnp.float32) * d0
    xs1 = jnp.dot(xb, wm1_s[...], preferred_element_type=jnp.float32) * d1
    for q in range(NQ):
        a_refs[q][...] = xs0[:, q * QW:(q + 1) * QW]
        a_refs[NQ + q][...] = xs1[:, q * QW:(q + 1) * QW]


def _xws(xt0, tv0, wiht0, whht0, bih0, bhh0, w00,
         xt1, tv1, wiht1, whht1, bih1, bhh1, w01,
         xp, deg0c, deg1c):
    nblk = ACC_ROWS // _RB
    quarter = jax.ShapeDtypeStruct((ACC_ROWS, QW), jnp.float32)
    inv = pl.BlockSpec((256, FEAT), lambda i: (0, 0))
    inv1 = pl.BlockSpec((256, 1), lambda i: (0, 0))
    invw = pl.BlockSpec((FEAT, 3 * FEAT), lambda i: (0, 0))
    invb = pl.BlockSpec((3 * FEAT,), lambda i: (0,))
    return pl.pallas_call(
        functools.partial(_k3_body),
        grid=(nblk,),
        in_specs=[inv, inv1, invw, invw, invb, invb, inv,
                  inv, inv1, invw, invw, invb, invb, inv,
                  pl.BlockSpec((_RB, FEAT), lambda i: (i, 0)),
                  pl.BlockSpec((_RB, 1), lambda i: (i, 0)),
                  pl.BlockSpec((_RB, 1), lambda i: (i, 0))],
        out_specs=(pl.BlockSpec((_RB, QW), lambda i: (i, 0)),) * (2 * NQ),
        out_shape=(quarter,) * (2 * NQ),
        scratch_shapes=[pltpu.VMEM((FEAT, FEAT), jnp.float32),
                        pltpu.VMEM((FEAT, FEAT), jnp.float32)],
    )(xt0, tv0, wiht0, whht0, bih0, bhh0, w00,
      xt1, tv1, wiht1, whht1, bih1, bhh1, w01, xp, deg0c, deg1c)


def _k4_body(src0_ref, dst0_ref, src1_ref, dst1_ref, *refs):
    a_refs = refs[:2 * NQ]
    o_refs = refs[2 * NQ:4 * NQ]
    sidx_v, didx_v, rows0_v, rows1_v, zbuf_v, acc_sh, sem0, sem1 = refs[4 * NQ:]
    cid = lax.axis_index("c")
    sid = lax.axis_index("s")

    z16 = jnp.zeros((16,), jnp.float32)

    def zb(i, _):
        for k in range(QW // 16):
            zbuf_v[i, pl.ds(k * 16, 16)] = z16
        return 0
    lax.fori_loop(0, 64, zb, 0)

    @pl.when(cid == 0)
    def _():
        pltpu.sync_copy(src0_ref.at[sid], sidx_v)
        pltpu.sync_copy(dst0_ref.at[sid], didx_v)

    @pl.when(cid == 1)
    def _():
        pltpu.sync_copy(src1_ref.at[sid], sidx_v)
        pltpu.sync_copy(dst1_ref.at[sid], didx_v)

    def do_quarter(a_ref, o_ref):
        for j in range(DEG_PER_TILE // 64):
            pltpu.sync_copy(
                zbuf_v,
                acc_sh.at[pl.ds(sid * DEG_PER_TILE + j * 64, 64)])
        plsc.subcore_barrier()

        def sl(v, j):
            return v.at[pl.ds(j * E_CHUNK, E_CHUNK)]

        pltpu.async_copy(a_ref.at[sl(sidx_v, 0)], rows0_v, sem0)

        def step(j, _):
            @pl.when(j % 2 == 0)
            def _():
                pltpu.make_async_copy(
                    a_ref.at[sl(sidx_v, j)], rows0_v, sem0).wait()

                @pl.when(j < E_CHUNKS - 1)
                def _():
                    pltpu.async_copy(
                        a_ref.at[sl(sidx_v, j + 1)], rows1_v, sem1)
                pltpu.sync_copy(rows0_v, acc_sh.at[sl(didx_v, j)], add=True)

            @pl.when(j % 2 == 1)
            def _():
                pltpu.make_async_copy(
                    a_ref.at[sl(sidx_v, j)], rows1_v, sem1).wait()

                @pl.when(j < E_CHUNKS - 1)
                def _():
                    pltpu.async_copy(
                        a_ref.at[sl(sidx_v, j + 1)], rows0_v, sem0)
                pltpu.sync_copy(rows1_v, acc_sh.at[sl(didx_v, j)], add=True)
            return 0
        lax.fori_loop(0, E_CHUNKS, step, 0)
        plsc.subcore_barrier()

        for j in range(DEG_PER_TILE // 128):
            s = pl.ds(sid * DEG_PER_TILE + j * 128, 128)
            pltpu.sync_copy(acc_sh.at[s], o_ref.at[s])
        plsc.subcore_barrier()

    @pl.when(cid == 0)
    def _():
        for q in range(NQ):
            do_quarter(a_refs[q], o_refs[q])

    @pl.when(cid == 1)
    def _():
        for q in range(NQ):
            do_quarter(a_refs[NQ + q], o_refs[NQ + q])


def _message_passing(src0p, dst0p, src1p, dst1p, aq):
    acc = jax.ShapeDtypeStruct((ACC_ROWS, QW), jnp.float32)
    f = functools.partial(
        pl.kernel,
        out_type=(acc,) * (2 * NQ),
        mesh=_mesh(),
        scratch_types=[
            pltpu.VMEM((E_PER_TILE,), jnp.int32),
            pltpu.VMEM((E_PER_TILE,), jnp.int32),
            pltpu.VMEM((E_CHUNK, QW), jnp.float32),
            pltpu.VMEM((E_CHUNK, QW), jnp.float32),
            pltpu.VMEM((64, QW), jnp.float32),
            pltpu.VMEM_SHARED((ACC_ROWS, QW), jnp.float32),
            pltpu.SemaphoreType.DMA,
            pltpu.SemaphoreType.DMA,
        ],
        compiler_params=pltpu.CompilerParams(use_tc_tiling_on_sc=False, needs_layout_passes=False),
    )
    return f(_k4_body)(src0p, dst0p, src1p, dst1p, *aq)


def _k5_body(*refs):
    o_refs = refs[:2 * NQ]
    a_refs = refs[2 * NQ:4 * NQ]
    deg0_ref, deg1_ref, wpt_ref, h_ref, hw_ref = refs[4 * NQ:]
    d0 = lax.rsqrt(deg0_ref[...] + 1.0)
    d1 = lax.rsqrt(deg1_ref[...] + 1.0)
    parts = []
    for q in range(NQ):
        parts.append(d0 * (o_refs[q][...] + a_refs[q][...])
                     + d1 * (o_refs[NQ + q][...] + a_refs[NQ + q][...]))
    h = jnp.maximum(jnp.concatenate(parts, axis=1), 0.0)
    wsum = wpt_ref[0:1, :] + wpt_ref[1:2, :]
    h_ref[...] = h
    hw_ref[...] = h * wsum


def _combine(oq, aq, deg0c, deg1c, wpostt):
    nblk = ACC_ROWS // _RB
    ob = pl.BlockSpec((_RB, QW), lambda i: (i, 0))
    return pl.pallas_call(
        _k5_body,
        grid=(nblk,),
        in_specs=[ob] * (4 * NQ)
        + [pl.BlockSpec((_RB, 1), lambda i: (i, 0)),
           pl.BlockSpec((_RB, 1), lambda i: (i, 0)),
           pl.BlockSpec((2, FEAT), lambda i: (0, 0))],
        out_specs=(pl.BlockSpec((_RB, FEAT), lambda i: (i, 0)),
                   pl.BlockSpec((_RB, FEAT), lambda i: (i, 0))),
        out_shape=(jax.ShapeDtypeStruct((ACC_ROWS, FEAT), jnp.float32),
                   jax.ShapeDtypeStruct((ACC_ROWS, FEAT), jnp.float32)),
    )(*oq, *aq, deg0c, deg1c, wpostt)


def _k6_body(h_ref, hw_ref, la_ref, lb_ref, bp_ref, res_ref,
             ia_v, ib_v, ga_v, gb_v, res_v, bp_v, sem):
    cid = lax.axis_index("c")
    sid = lax.axis_index("s")
    w = sid * NC + cid

    pltpu.sync_copy(la_ref.at[w], ia_v)
    pltpu.sync_copy(lb_ref.at[w], ib_v)
    pltpu.sync_copy(bp_ref, bp_v)
    c = jnp.sum(bp_v[...])
    i16 = lax.iota(jnp.int32, 16)

    for j in range(L_CHUNKS):
        pltpu.async_copy(h_ref.at[ia_v.at[j]], ga_v, sem).wait()
        pltpu.async_copy(hw_ref.at[ib_v.at[j]], gb_v, sem).wait()

        def grp(g, _):
            vec = jnp.zeros((16,), jnp.float32)
            for i in range(16):
                p = g * 16 + i
                acc = jnp.zeros((16,), jnp.float32)
                for k in range(16):
                    acc = acc + (ga_v[p, pl.ds(k * 16, 16)]
                                 * gb_v[p, pl.ds(k * 16, 16)])
                s = jnp.sum(acc)
                vec = jnp.where(i16 == i, s, vec)
            res_v[pl.ds(j * 128 + g * 16, 16)] = vec + c
            return 0
        lax.fori_loop(0, 8, grp, 0)

    pltpu.sync_copy(res_v, res_ref.at[w])


def _label_score(h, hw, la, lb, bp16):
    f = functools.partial(
        pl.kernel,
        out_type=jax.ShapeDtypeStruct((NW, L_PER_W), jnp.float32),
        mesh=_mesh(),
        scratch_types=[
            pltpu.VMEM((L_CHUNKS, 128), jnp.int32),
            pltpu.VMEM((L_CHUNKS, 128), jnp.int32),
            pltpu.VMEM((128, FEAT), jnp.float32),
            pltpu.VMEM((128, FEAT), jnp.float32),
            pltpu.VMEM((L_PER_W,), jnp.float32),
            pltpu.VMEM((16,), jnp.float32),
            pltpu.SemaphoreType.DMA,
        ],
        compiler_params=pltpu.CompilerParams(needs_layout_passes=False),
    )
    return f(_k6_body)(h, hw, la, lb, bp16)


def kernel(x, edge_index_rel0, edge_index_rel1, edge_label_index,
           p0, Wih0, Whh0, bih0, bhh0, W0,
           p1, Wih1, Whh1, bih1, bhh1, W1,
           Wpost, bpost):
    p128 = jnp.pad(jnp.stack([p0, p1], axis=1), ((0, 0), (0, 126)))
    p2 = jnp.stack([p0, p1], axis=1)

    npad = E_PAD - N_EDGES
    ar = lax.iota(jnp.int32, npad)
    sent_src = ar % 64
    sent_dst = N_NODES + (ar % 128)

    def prep_edges(ei):
        s = jnp.concatenate([ei[0], sent_src]).reshape(NS, E_PER_TILE)
        d = jnp.concatenate([ei[1], sent_dst]).reshape(NS, E_PER_TILE)
        return s, d

    src0p, dst0p = prep_edges(edge_index_rel0)
    src1p, dst1p = prep_edges(edge_index_rel1)

    lpad = L_PAD - N_LABEL
    lar = lax.iota(jnp.int32, lpad) % 64
    la = jnp.concatenate([edge_label_index[0], lar]).reshape(NW, L_CHUNKS, 128)
    lb = jnp.concatenate([edge_label_index[1], lar]).reshape(NW, L_CHUNKS, 128)

    wiht0, whht0 = Wih0.T, Whh0.T
    wiht1, whht1 = Wih1.T, Whh1.T
    wpostt = Wpost.T
    bp16 = jnp.pad(bpost, (0, 14))

    S = _scores(x, p128)
    spad = jnp.pad(S[:, :2], ((0, 240), (0, 0)),
                   constant_values=-jnp.inf)
    s0 = spad[:, 0].reshape(80, 128)
    s1 = spad[:, 1].reshape(80, 128)
    idx0, idx1, tv0, tv1 = _topk(s0, s1, p2)

    deg0, deg1, xt0, xt1 = _deg_and_gather(dst0p, dst1p, idx0, idx1, x)
    deg0c = deg0.reshape(ACC_ROWS, 1)
    deg1c = deg1.reshape(ACC_ROWS, 1)

    xp = jnp.pad(x, ((0, ACC_ROWS - N_NODES), (0, 0)))
    aq = _xws(xt0, tv0.reshape(256, 1), wiht0, whht0, bih0, bhh0, W0,
              xt1, tv1.reshape(256, 1), wiht1, whht1, bih1, bhh1, W1,
              xp, deg0c, deg1c)

    oq = _message_passing(src0p, dst0p, src1p, dst1p, aq)

    h, hw = _combine(oq, aq, deg0c, deg1c, wpostt)

    res = _label_score(h, hw, la, lb, bp16)
    return res.reshape(L_PAD)[:N_LABEL]


# speedup vs baseline: 12.2293x; 1.0327x over previous
import functools

import jax
import jax.numpy as jnp
from jax import lax
from jax.experimental import pallas as pl
from jax.experimental.pallas import tpu as pltpu
from jax.experimental.pallas import tpu_sc as plsc

N_NODES = 10000
FEAT = 256
N_EDGES = 160000
N_LABEL = 20000

NC = 2
NS = 16
NW = NC * NS

E_CHUNK = 512
E_CHUNKS = 20
E_PER_TILE = E_CHUNKS * E_CHUNK
E_PAD = NS * E_PER_TILE
ACC_ROWS = 10240
DEG_PER_TILE = ACC_ROWS // NS

L_CHUNKS = 5
L_PER_W = L_CHUNKS * 128
L_PAD = NW * L_PER_W

def _mesh():
    return plsc.VectorSubcoreMesh(core_axis_name="c", subcore_axis_name="s")


def _k1_body(x_ref, p_ref, s_ref):
    s_ref[...] = jnp.dot(x_ref[...], p_ref[...],
                         preferred_element_type=jnp.float32)


def _scores(x, p128):
    return pl.pallas_call(
        _k1_body,
        out_shape=jax.ShapeDtypeStruct((N_NODES, 128), jnp.float32),
    )(x, p128)


def _k1b_body(s0_ref, s1_ref, p2_ref, idx0_ref, idx1_ref, tv0_ref, tv1_ref):
    s0 = s0_ref[...]
    s1 = s1_ref[...]
    pp = p2_ref[...]
    nrm0 = jnp.sqrt(jnp.sum(pp[:, 0:1] * pp[:, 0:1]))
    nrm1 = jnp.sqrt(jnp.sum(pp[:, 1:2] * pp[:, 1:2]))
    inv0 = 1.0 / (nrm0 + 1e-16)
    inv1 = 1.0 / (nrm1 + 1e-16)

    fi = (lax.broadcasted_iota(jnp.int32, (80, 128), 0) * 128
          + lax.broadcasted_iota(jnp.int32, (80, 128), 1))
    i256 = lax.iota(jnp.int32, 256)
    neg = jnp.float32(-jnp.inf)
    big = jnp.int32(1 << 30)

    def step(r, carry):
        c0, c1, va0, va1, ia0, ia1 = carry

        m0 = jnp.max(c0)
        ix0 = jnp.min(jnp.where(c0 == m0, fi, big))
        c0 = jnp.where(fi == ix0, neg, c0)
        va0 = jnp.where(i256 == r, m0, va0)
        ia0 = jnp.where(i256 == r, ix0, ia0)

        m1 = jnp.max(c1)
        ix1 = jnp.min(jnp.where(c1 == m1, fi, big))
        c1 = jnp.where(fi == ix1, neg, c1)
        va1 = jnp.where(i256 == r, m1, va1)
        ia1 = jnp.where(i256 == r, ix1, ia1)

        return c0, c1, va0, va1, ia0, ia1

    init = (s0, s1,
            jnp.zeros((256,), jnp.float32), jnp.zeros((256,), jnp.float32),
            jnp.zeros((256,), jnp.int32), jnp.zeros((256,), jnp.int32))
    _, _, va0, va1, ia0, ia1 = lax.fori_loop(0, 256, step, init)

    idx0_ref[...] = ia0
    idx1_ref[...] = ia1
    tv0_ref[...] = jnp.tanh(va0 * inv0)
    tv1_ref[...] = jnp.tanh(va1 * inv1)


def _topk(s0, s1, p2):
    return pl.pallas_call(
        _k1b_body,
        out_shape=(
            jax.ShapeDtypeStruct((256,), jnp.int32),
            jax.ShapeDtypeStruct((256,), jnp.int32),
            jax.ShapeDtypeStruct((256,), jnp.float32),
            jax.ShapeDtypeStruct((256,), jnp.float32),
        ),
    )(s0, s1, p2)


def _k2_body(dst0_ref, dst1_ref, idx0_ref, idx1_ref, x_ref,
             deg0_ref, deg1_ref, xt0_ref, xt1_ref,
             idx_v, ones_v, gidx_v, rows_v, zrow_v, deg_sh, sem):
    cid = lax.axis_index("c")
    sid = lax.axis_index("s")

    z16 = jnp.zeros((16,), jnp.float32)
    o16 = jnp.ones((16,), jnp.float32)

    def zb(k, _):
        zrow_v[pl.ds(k * 16, 16)] = z16
        return 0
    lax.fori_loop(0, DEG_PER_TILE // 16, zb, 0)

    def ob(k, _):
        ones_v[pl.ds(k * 16, 16)] = o16
        return 0
    lax.fori_loop(0, E_CHUNK // 16, ob, 0)

    pltpu.sync_copy(zrow_v, deg_sh.at[pl.ds(sid * DEG_PER_TILE, DEG_PER_TILE)])
    plsc.subcore_barrier()

    @pl.when(cid == 0)
    def _():
        pltpu.sync_copy(dst0_ref.at[sid], idx_v)

    @pl.when(cid == 1)
    def _():
        pltpu.sync_copy(dst1_ref.at[sid], idx_v)

    def add1(j, _):
        pltpu.sync_copy(
            ones_v, deg_sh.at[idx_v.at[pl.ds(j * E_CHUNK, E_CHUNK)]], add=True)
        return 0
    lax.fori_loop(0, E_CHUNKS, add1, 0)
    plsc.subcore_barrier()

    @pl.when(cid == 0)
    def _():
        pltpu.sync_copy(deg_sh.at[pl.ds(sid * DEG_PER_TILE, DEG_PER_TILE)],
                        deg0_ref.at[pl.ds(sid * DEG_PER_TILE, DEG_PER_TILE)])

    @pl.when(cid == 1)
    def _():
        pltpu.sync_copy(deg_sh.at[pl.ds(sid * DEG_PER_TILE, DEG_PER_TILE)],
                        deg1_ref.at[pl.ds(sid * DEG_PER_TILE, DEG_PER_TILE)])

    @pl.when(cid == 0)
    def _():
        pltpu.sync_copy(idx0_ref.at[pl.ds(sid * 16, 16)], gidx_v)
        pltpu.async_copy(x_ref.at[gidx_v], rows_v, sem).wait()
        pltpu.sync_copy(rows_v, xt0_ref.at[pl.ds(sid * 16, 16)])

    @pl.when(cid == 1)
    def _():
        pltpu.sync_copy(idx1_ref.at[pl.ds(sid * 16, 16)], gidx_v)
        pltpu.async_copy(x_ref.at[gidx_v], rows_v, sem).wait()
        pltpu.sync_copy(rows_v, xt1_ref.at[pl.ds(sid * 16, 16)])


def _deg_and_gather(dst0p, dst1p, idx0, idx1, x):
    f = functools.partial(
        pl.kernel,
        out_type=(
            jax.ShapeDtypeStruct((ACC_ROWS,), jnp.float32),
            jax.ShapeDtypeStruct((ACC_ROWS,), jnp.float32),
            jax.ShapeDtypeStruct((256, FEAT), jnp.float32),
            jax.ShapeDtypeStruct((256, FEAT), jnp.float32),
        ),
        mesh=_mesh(),
        scratch_types=[
            pltpu.VMEM((E_PER_TILE,), jnp.int32),
            pltpu.VMEM((E_CHUNK,), jnp.float32),
            pltpu.VMEM((16,), jnp.int32),
            pltpu.VMEM((16, FEAT), jnp.float32),
            pltpu.VMEM((DEG_PER_TILE,), jnp.float32),
            pltpu.VMEM_SHARED((ACC_ROWS,), jnp.float32),
            pltpu.SemaphoreType.DMA,
        ],
    )
    return f(_k2_body)(dst0p, dst1p, idx0, idx1, x)


def _gru_half(xt, tv, wiht, whht, bih, bhh, w0):
    xts = xt * tv
    gi = jnp.dot(xts, wiht, preferred_element_type=jnp.float32) + bih[None, :]
    gh = jnp.dot(w0, whht, preferred_element_type=jnp.float32) + bhh[None, :]
    i_r, i_z, i_n = gi[:, :256], gi[:, 256:512], gi[:, 512:]
    h_r, h_z, h_n = gh[:, :256], gh[:, 256:512], gh[:, 512:]
    r = 1.0 / (1.0 + jnp.exp(-(i_r + h_r)))
    z = 1.0 / (1.0 + jnp.exp(-(i_z + h_z)))
    n = jnp.tanh(i_n + r * h_n)
    return (1.0 - z) * n + z * w0


_RB = ACC_ROWS // 8
NQ = 4
QW = FEAT // NQ


def _k3_body(xt0_ref, tv0_ref, wiht0_ref, whht0_ref, bih0_ref, bhh0_ref,
             w00_ref, xt1_ref, tv1_ref, wiht1_ref, whht1_ref, bih1_ref,
             bhh1_ref, w01_ref, x_ref, deg0_ref, deg1_ref, *rest):
    a_refs = rest[:2 * NQ]
    wm0_s, wm1_s = rest[2 * NQ:]
    @pl.when(pl.program_id(0) == 0)
    def _():
        wm0_s[...] = _gru_half(xt0_ref[...], tv0_ref[...], wiht0_ref[...],
                               whht0_ref[...], bih0_ref[...], bhh0_ref[...],
                               w00_ref[...])
        wm1_s[...] = _gru_half(xt1_ref[...], tv1_ref[...], wiht1_ref[...],
                               whht1_ref[...], bih1_ref[...], bhh1_ref[...],
                               w01_ref[...])
    xb = x_ref[...]
    d0 = lax.rsqrt(deg0_ref[...] + 1.0)
    d1 = lax.rsqrt(deg1_ref[...] + 1.0)
    xs0 = jnp.dot(xb, wm0_s[...], preferred_element_type=jnp.float32) * d0
    xs1 = jnp.dot(xb, wm1_s[...], preferred_element_type=jnp.float32) * d1
    for q in range(NQ):
        a_refs[q][...] = xs0[:, q * QW:(q + 1) * QW]
        a_refs[NQ + q][...] = xs1[:, q * QW:(q + 1) * QW]


def _xws(xt0, tv0, wiht0, whht0, bih0, bhh0, w00,
         xt1, tv1, wiht1, whht1, bih1, bhh1, w01,
         xp, deg0c, deg1c):
    nblk = ACC_ROWS // _RB
    quarter = jax.ShapeDtypeStruct((ACC_ROWS, QW), jnp.float32)
    inv = pl.BlockSpec((256, FEAT), lambda i: (0, 0))
    inv1 = pl.BlockSpec((256, 1), lambda i: (0, 0))
    invw = pl.BlockSpec((FEAT, 3 * FEAT), lambda i: (0, 0))
    invb = pl.BlockSpec((3 * FEAT,), lambda i: (0,))
    return pl.pallas_call(
        functools.partial(_k3_body),
        grid=(nblk,),
        in_specs=[inv, inv1, invw, invw, invb, invb, inv,
                  inv, inv1, invw, invw, invb, invb, inv,
                  pl.BlockSpec((_RB, FEAT), lambda i: (i, 0)),
                  pl.BlockSpec((_RB, 1), lambda i: (i, 0)),
                  pl.BlockSpec((_RB, 1), lambda i: (i, 0))],
        out_specs=(pl.BlockSpec((_RB, QW), lambda i: (i, 0)),) * (2 * NQ),
        out_shape=(quarter,) * (2 * NQ),
        scratch_shapes=[pltpu.VMEM((FEAT, FEAT), jnp.float32),
                        pltpu.VMEM((FEAT, FEAT), jnp.float32)],
    )(xt0, tv0, wiht0, whht0, bih0, bhh0, w00,
      xt1, tv1, wiht1, whht1, bih1, bhh1, w01, xp, deg0c, deg1c)


def _k4_body(src0_ref, dst0_ref, src1_ref, dst1_ref, *refs):
    a_refs = refs[:2 * NQ]
    o_refs = refs[2 * NQ:4 * NQ]
    sidx_v, didx_v, rows0_v, rows1_v, zbuf_v, acc_sh, sem0, sem1 = refs[4 * NQ:]
    cid = lax.axis_index("c")
    sid = lax.axis_index("s")

    z16 = jnp.zeros((16,), jnp.float32)

    def zb(i, _):
        for k in range(QW // 16):
            zbuf_v[i, pl.ds(k * 16, 16)] = z16
        return 0
    lax.fori_loop(0, 64, zb, 0)

    @pl.when(cid == 0)
    def _():
        pltpu.sync_copy(src0_ref.at[sid], sidx_v)
        pltpu.sync_copy(dst0_ref.at[sid], didx_v)

    @pl.when(cid == 1)
    def _():
        pltpu.sync_copy(src1_ref.at[sid], sidx_v)
        pltpu.sync_copy(dst1_ref.at[sid], didx_v)

    def do_quarter(a_ref, o_ref):
        for j in range(DEG_PER_TILE // 64):
            pltpu.sync_copy(
                zbuf_v,
                acc_sh.at[pl.ds(sid * DEG_PER_TILE + j * 64, 64)])
        plsc.subcore_barrier()

        def sl(v, j):
            return v.at[pl.ds(j * E_CHUNK, E_CHUNK)]

        pltpu.async_copy(a_ref.at[sl(sidx_v, 0)], rows0_v, sem0)

        def step(j, _):
            @pl.when(j % 2 == 0)
            def _():
                pltpu.make_async_copy(
                    a_ref.at[sl(sidx_v, j)], rows0_v, sem0).wait()

                @pl.when(j < E_CHUNKS - 1)
                def _():
                    pltpu.async_copy(
                        a_ref.at[sl(sidx_v, j + 1)], rows1_v, sem1)
                pltpu.sync_copy(rows0_v, acc_sh.at[sl(didx_v, j)], add=True)

            @pl.when(j % 2 == 1)
            def _():
                pltpu.make_async_copy(
                    a_ref.at[sl(sidx_v, j)], rows1_v, sem1).wait()

                @pl.when(j < E_CHUNKS - 1)
                def _():
                    pltpu.async_copy(
                        a_ref.at[sl(sidx_v, j + 1)], rows0_v, sem0)
                pltpu.sync_copy(rows1_v, acc_sh.at[sl(didx_v, j)], add=True)
            return 0
        lax.fori_loop(0, E_CHUNKS, step, 0)
        plsc.subcore_barrier()

        for j in range(DEG_PER_TILE // 128):
            s = pl.ds(sid * DEG_PER_TILE + j * 128, 128)
            pltpu.sync_copy(acc_sh.at[s], o_ref.at[s])
        plsc.subcore_barrier()

    @pl.when(cid == 0)
    def _():
        for q in range(NQ):
            do_quarter(a_refs[q], o_refs[q])

    @pl.when(cid == 1)
    def _():
        for q in range(NQ):
            do_quarter(a_refs[NQ + q], o_refs[NQ + q])


def _message_passing(src0p, dst0p, src1p, dst1p, aq):
    acc = jax.ShapeDtypeStruct((ACC_ROWS, QW), jnp.float32)
    f = functools.partial(
        pl.kernel,
        out_type=(acc,) * (2 * NQ),
        mesh=_mesh(),
        scratch_types=[
            pltpu.VMEM((E_PER_TILE,), jnp.int32),
            pltpu.VMEM((E_PER_TILE,), jnp.int32),
            pltpu.VMEM((E_CHUNK, QW), jnp.float32),
            pltpu.VMEM((E_CHUNK, QW), jnp.float32),
            pltpu.VMEM((64, QW), jnp.float32),
            pltpu.VMEM_SHARED((ACC_ROWS, QW), jnp.float32),
            pltpu.SemaphoreType.DMA,
            pltpu.SemaphoreType.DMA,
        ],
        compiler_params=pltpu.CompilerParams(use_tc_tiling_on_sc=False, needs_layout_passes=False),
    )
    return f(_k4_body)(src0p, dst0p, src1p, dst1p, *aq)


def _k5_body(*refs):
    o_refs = refs[:2 * NQ]
    a_refs = refs[2 * NQ:4 * NQ]
    deg0_ref, deg1_ref, wpt_ref, h_ref, hw_ref = refs[4 * NQ:]
    d0 = lax.rsqrt(deg0_ref[...] + 1.0)
    d1 = lax.rsqrt(deg1_ref[...] + 1.0)
    parts = []
    for q in range(NQ):
        parts.append(d0 * (o_refs[q][...] + a_refs[q][...])
                     + d1 * (o_refs[NQ + q][...] + a_refs[NQ + q][...]))
    h = jnp.maximum(jnp.concatenate(parts, axis=1), 0.0)
    wsum = wpt_ref[0:1, :] + wpt_ref[1:2, :]
    h_ref[...] = h
    hw_ref[...] = h * wsum


def _combine(oq, aq, deg0c, deg1c, wpostt):
    nblk = ACC_ROWS // _RB
    ob = pl.BlockSpec((_RB, QW), lambda i: (i, 0))
    return pl.pallas_call(
        _k5_body,
        grid=(nblk,),
        in_specs=[ob] * (4 * NQ)
        + [pl.BlockSpec((_RB, 1), lambda i: (i, 0)),
           pl.BlockSpec((_RB, 1), lambda i: (i, 0)),
           pl.BlockSpec((2, FEAT), lambda i: (0, 0))],
        out_specs=(pl.BlockSpec((_RB, FEAT), lambda i: (i, 0)),
                   pl.BlockSpec((_RB, FEAT), lambda i: (i, 0))),
        out_shape=(jax.ShapeDtypeStruct((ACC_ROWS, FEAT), jnp.float32),
                   jax.ShapeDtypeStruct((ACC_ROWS, FEAT), jnp.float32)),
    )(*oq, *aq, deg0c, deg1c, wpostt)


L_CH = 64
L_NC = L_PER_W // L_CH


def _k6_body(h_ref, hw_ref, la_ref, lb_ref, bp_ref, res_ref,
             ia_v, ib_v, ga0_v, gb0_v, ga1_v, gb1_v, res_v, bp_v,
             sa0, sb0, sa1, sb1):
    cid = lax.axis_index("c")
    sid = lax.axis_index("s")
    w = sid * NC + cid

    pltpu.sync_copy(la_ref.at[w], ia_v)
    pltpu.sync_copy(lb_ref.at[w], ib_v)
    pltpu.sync_copy(bp_ref, bp_v)
    c = jnp.sum(bp_v[...])
    i16 = lax.iota(jnp.int32, 16)

    def isl(v, j):
        return v.at[pl.ds(j * L_CH, L_CH)]

    def fetch(j, ga, gb, sa, sb):
        pltpu.async_copy(h_ref.at[isl(ia_v, j)], ga, sa)
        pltpu.async_copy(hw_ref.at[isl(ib_v, j)], gb, sb)

    def wait(j, ga, gb, sa, sb):
        pltpu.make_async_copy(h_ref.at[isl(ia_v, j)], ga, sa).wait()
        pltpu.make_async_copy(hw_ref.at[isl(ib_v, j)], gb, sb).wait()

    def compute(j, ga_v, gb_v):
        def grp(g, _):
            vec = jnp.zeros((16,), jnp.float32)
            for i in range(16):
                p = g * 16 + i
                acc = jnp.zeros((16,), jnp.float32)
                for k in range(16):
                    acc = acc + (ga_v[p, pl.ds(k * 16, 16)]
                                 * gb_v[p, pl.ds(k * 16, 16)])
                s = jnp.sum(acc)
                vec = jnp.where(i16 == i, s, vec)
            res_v[pl.ds(j * L_CH + g * 16, 16)] = vec + c
            return 0
        lax.fori_loop(0, L_CH // 16, grp, 0)

    fetch(0, ga0_v, gb0_v, sa0, sb0)

    def step(j, _):
        @pl.when(j % 2 == 0)
        def _():
            wait(j, ga0_v, gb0_v, sa0, sb0)

            @pl.when(j < L_NC - 1)
            def _():
                fetch(j + 1, ga1_v, gb1_v, sa1, sb1)
            compute(j, ga0_v, gb0_v)

        @pl.when(j % 2 == 1)
        def _():
            wait(j, ga1_v, gb1_v, sa1, sb1)

            @pl.when(j < L_NC - 1)
            def _():
                fetch(j + 1, ga0_v, gb0_v, sa0, sb0)
            compute(j, ga1_v, gb1_v)
        return 0
    lax.fori_loop(0, L_NC, step, 0)

    pltpu.sync_copy(res_v, res_ref.at[w])


def _label_score(h, hw, la, lb, bp16):
    f = functools.partial(
        pl.kernel,
        out_type=jax.ShapeDtypeStruct((NW, L_PER_W), jnp.float32),
        mesh=_mesh(),
        scratch_types=[
            pltpu.VMEM((L_PER_W,), jnp.int32),
            pltpu.VMEM((L_PER_W,), jnp.int32),
            pltpu.VMEM((L_CH, FEAT), jnp.float32),
            pltpu.VMEM((L_CH, FEAT), jnp.float32),
            pltpu.VMEM((L_CH, FEAT), jnp.float32),
            pltpu.VMEM((L_CH, FEAT), jnp.float32),
            pltpu.VMEM((L_PER_W,), jnp.float32),
            pltpu.VMEM((16,), jnp.float32),
            pltpu.SemaphoreType.DMA,
            pltpu.SemaphoreType.DMA,
            pltpu.SemaphoreType.DMA,
            pltpu.SemaphoreType.DMA,
        ],
        compiler_params=pltpu.CompilerParams(needs_layout_passes=False),
    )
    return f(_k6_body)(h, hw, la, lb, bp16)


def kernel(x, edge_index_rel0, edge_index_rel1, edge_label_index,
           p0, Wih0, Whh0, bih0, bhh0, W0,
           p1, Wih1, Whh1, bih1, bhh1, W1,
           Wpost, bpost):
    p128 = jnp.pad(jnp.stack([p0, p1], axis=1), ((0, 0), (0, 126)))
    p2 = jnp.stack([p0, p1], axis=1)

    npad = E_PAD - N_EDGES
    ar = lax.iota(jnp.int32, npad)
    sent_src = ar % 64
    sent_dst = N_NODES + (ar % 128)

    def prep_edges(ei):
        s = jnp.concatenate([ei[0], sent_src]).reshape(NS, E_PER_TILE)
        d = jnp.concatenate([ei[1], sent_dst]).reshape(NS, E_PER_TILE)
        return s, d

    src0p, dst0p = prep_edges(edge_index_rel0)
    src1p, dst1p = prep_edges(edge_index_rel1)

    lpad = L_PAD - N_LABEL
    lar = lax.iota(jnp.int32, lpad) % 64
    la = jnp.concatenate([edge_label_index[0], lar]).reshape(NW, L_PER_W)
    lb = jnp.concatenate([edge_label_index[1], lar]).reshape(NW, L_PER_W)

    wiht0, whht0 = Wih0.T, Whh0.T
    wiht1, whht1 = Wih1.T, Whh1.T
    wpostt = Wpost.T
    bp16 = jnp.pad(bpost, (0, 14))

    S = _scores(x, p128)
    spad = jnp.pad(S[:, :2], ((0, 240), (0, 0)),
                   constant_values=-jnp.inf)
    s0 = spad[:, 0].reshape(80, 128)
    s1 = spad[:, 1].reshape(80, 128)
    idx0, idx1, tv0, tv1 = _topk(s0, s1, p2)

    deg0, deg1, xt0, xt1 = _deg_and_gather(dst0p, dst1p, idx0, idx1, x)
    deg0c = deg0.reshape(ACC_ROWS, 1)
    deg1c = deg1.reshape(ACC_ROWS, 1)

    xp = jnp.pad(x, ((0, ACC_ROWS - N_NODES), (0, 0)))
    aq = _xws(xt0, tv0.reshape(256, 1), wiht0, whht0, bih0, bhh0, W0,
              xt1, tv1.reshape(256, 1), wiht1, whht1, bih1, bhh1, W1,
              xp, deg0c, deg1c)

    oq = _message_passing(src0p, dst0p, src1p, dst1p, aq)

    h, hw = _combine(oq, aq, deg0c, deg1c, wpostt)

    res = _label_score(h, hw, la, lb, bp16)
    return res.reshape(L_PAD)[:N_LABEL]
